# baseline (speedup 1.0000x reference)
import functools

import jax
import jax.numpy as jnp
from jax import lax
from jax.experimental import pallas as pl
from jax.experimental.pallas import tpu as pltpu
from jax.experimental.pallas import tpu_sc as plsc

_NC, _NS = 2, 16
_NW = _NC * _NS


def _topk_body(vr_ref, vt_ref, idx_ref, d_ref, *, K1, V):
    b = pl.program_id(0)
    vr = vr_ref[0]
    vt = vt_ref[0]
    R = vr.shape[0]
    inner = (vr[:, 0:1] * vt[0:1, :] + vr[:, 1:2] * vt[1:2, :]
             + vr[:, 2:3] * vt[2:3, :])
    qc = jnp.sum(vt * vt, axis=0, keepdims=True)
    qr = jnp.sum(vr * vr, axis=1, keepdims=True)
    neg = 2.0 * inner - qc - qr
    iota = lax.broadcasted_iota(jnp.int32, (R, V), 1)
    iok = lax.broadcasted_iota(jnp.int32, (R, K1), 1)
    negbig = jnp.float32(-3e38)

    def step(t, carry):
        neg, ai, ad = carry
        m = jnp.max(neg, axis=1, keepdims=True)
        sel = jnp.where(neg == m, iota, V)
        j = jnp.min(sel, axis=1, keepdims=True)
        ai = jnp.where(iok == t, j, ai)
        ad = jnp.where(iok == t, -m, ad)
        neg = jnp.where(iota == j, negbig, neg)
        return neg, ai, ad

    ai0 = jnp.zeros((R, K1), jnp.int32)
    ad0 = jnp.zeros((R, K1), jnp.float32)
    _, ai, ad = lax.fori_loop(0, K1, step, (neg, ai0, ad0))
    idx_ref[0] = ai + b * V
    d_ref[0] = ad


def _topk(verts, K1, R):
    bs, V, _ = verts.shape
    vt = jnp.transpose(verts, (0, 2, 1))
    return pl.pallas_call(
        functools.partial(_topk_body, K1=K1, V=V),
        grid=(bs, V // R),
        in_specs=[pl.BlockSpec((1, R, 3), lambda b, i: (b, i, 0)),
                  pl.BlockSpec((1, 3, V), lambda b, i: (b, 0, 0))],
        out_specs=[pl.BlockSpec((1, R, K1), lambda b, i: (b, i, 0)),
                   pl.BlockSpec((1, R, K1), lambda b, i: (b, i, 0))],
        out_shape=[jax.ShapeDtypeStruct((bs, V, K1), jnp.int32),
                   jax.ShapeDtypeStruct((bs, V, K1), jnp.float32)],
    )(verts, vt)


def _mm_body(x_ref, w_ref, b_ref, o_ref, *, act, K):
    x = x_ref[...]
    w = w_ref[...]
    if K <= 4:
        acc = x[:, 0:1] * w[0:1, :]
        for k in range(1, K):
            acc = acc + x[:, k:k + 1] * w[k:k + 1, :]
    else:
        acc = jnp.dot(x, w, preferred_element_type=jnp.float32)
    acc = acc + b_ref[...]
    if act == "relu":
        acc = jnp.maximum(acc, 0.0)
    o_ref[...] = acc


def _mm(x, w, b=None, act=None):
    M, K = x.shape
    N = w.shape[1]
    if b is None:
        b = jnp.zeros((N,), jnp.float32)
    BM = min(1024, M)
    return pl.pallas_call(
        functools.partial(_mm_body, act=act, K=K),
        grid=(M // BM,),
        in_specs=[pl.BlockSpec((BM, K), lambda i: (i, 0)),
                  pl.BlockSpec((K, N), lambda i: (0, 0)),
                  pl.BlockSpec((1, N), lambda i: (0, 0))],
        out_specs=pl.BlockSpec((BM, N), lambda i: (i, 0)),
        out_shape=jax.ShapeDtypeStruct((M, N), jnp.float32),
    )(x, w, b[None, :])


def _bn_body(x_ref, g_ref, b_ref, o_ref, *, relu):
    x = x_ref[...]
    m = jnp.mean(x, axis=0, keepdims=True)
    v = jnp.mean((x - m) ** 2, axis=0, keepdims=True)
    y = g_ref[...] * (x - m) / jnp.sqrt(v + 1e-5) + b_ref[...]
    if relu:
        y = jnp.maximum(y, 0.0)
    o_ref[...] = y


def _bn(x, g, b, relu=True):
    M, C = x.shape
    return pl.pallas_call(
        functools.partial(_bn_body, relu=relu),
        out_shape=jax.ShapeDtypeStruct((M, C), jnp.float32),
    )(x, g[None, :], b[None, :])


def _pick_bm(M, N, D):
    for bm in (1024, 512, 256, 128, 64, 32):
        if M % bm == 0 and bm * N * D * 4 <= 6 * 2**20:
            return bm
    return 32


def _rinv(r2):
    return 1.0 / jnp.maximum(jnp.sqrt(jnp.maximum(r2, 0.0)), 1e-12)


def _theta(xyz, xv, r2, sd):
    ri = _rinv(r2)
    d0 = xyz[:, :, 0] - xv[:, 0:1]
    d1 = xyz[:, :, 1] - xv[:, 1:2]
    d2 = xyz[:, :, 2] - xv[:, 2:3]
    dot = (d0[:, :, None] * sd[0][None, None, :]
           + d1[:, :, None] * sd[1][None, None, :]
           + d2[:, :, None] * sd[2][None, None, :])
    return jnp.maximum(dot * ri[:, :, None], 0.0)


def _comb_surface_body(xyz_ref, xv_ref, r2_ref, sd_ref, o_ref):
    th = _theta(xyz_ref[...], xv_ref[...], r2_ref[...], sd_ref[...])
    o_ref[...] = jnp.max(th, axis=1)


def _comb_surface(xyz, xv, r2, sd):
    M, N, W = xyz.shape
    C = sd.shape[1]
    BM = _pick_bm(M, N, W + C)
    return pl.pallas_call(
        _comb_surface_body,
        grid=(M // BM,),
        in_specs=[pl.BlockSpec((BM, N, W), lambda i: (i, 0, 0)),
                  pl.BlockSpec((BM, 4), lambda i: (i, 0)),
                  pl.BlockSpec((BM, N), lambda i: (i, 0)),
                  pl.BlockSpec((3, C), lambda i: (0, 0))],
        out_specs=pl.BlockSpec((BM, C), lambda i: (i, 0)),
        out_shape=jax.ShapeDtypeStruct((M, C), jnp.float32),
    )(xyz, xv, r2, sd)


def _comb_layer_body(fs_ref, xyz_ref, xv_ref, fc_ref, r2_ref, sd_ref, o_ref):
    th = _theta(xyz_ref[...], xv_ref[...], r2_ref[...], sd_ref[...])
    o_ref[...] = fc_ref[...] + jnp.max(th * fs_ref[...], axis=1)


def _comb_layer(fs, xyz, xv, fc, r2, sd):
    M, N, C = fs.shape
    W = xyz.shape[2]
    BM = _pick_bm(M, N, C + W + 8)
    return pl.pallas_call(
        _comb_layer_body,
        grid=(M // BM,),
        in_specs=[pl.BlockSpec((BM, N, C), lambda i: (i, 0, 0)),
                  pl.BlockSpec((BM, N, W), lambda i: (i, 0, 0)),
                  pl.BlockSpec((BM, 4), lambda i: (i, 0)),
                  pl.BlockSpec((BM, C), lambda i: (i, 0)),
                  pl.BlockSpec((BM, N), lambda i: (i, 0)),
                  pl.BlockSpec((3, C), lambda i: (0, 0))],
        out_specs=pl.BlockSpec((BM, C), lambda i: (i, 0)),
        out_shape=jax.ShapeDtypeStruct((M, C), jnp.float32),
    )(fs, xyz, xv, fc, r2, sd)


def _comb_layer_packed_body(g_ref, xv_ref, fc_ref, r2_ref, sd_ref, o_ref, *,
                            fsoff):
    x = g_ref[...]
    th = _theta(x, xv_ref[...], r2_ref[...], sd_ref[...])
    o_ref[...] = fc_ref[...] + jnp.max(th * x[:, :, fsoff:], axis=1)


def _comb_layer_packed(gath, xv, fc, r2, sd, fsoff):
    M, N, D = gath.shape
    C = D - fsoff
    BM = _pick_bm(M, N, D + C)
    return pl.pallas_call(
        functools.partial(_comb_layer_packed_body, fsoff=fsoff),
        grid=(M // BM,),
        in_specs=[pl.BlockSpec((BM, N, D), lambda i: (i, 0, 0)),
                  pl.BlockSpec((BM, 4), lambda i: (i, 0)),
                  pl.BlockSpec((BM, C), lambda i: (i, 0)),
                  pl.BlockSpec((BM, N), lambda i: (i, 0)),
                  pl.BlockSpec((3, C), lambda i: (0, 0))],
        out_specs=pl.BlockSpec((BM, C), lambda i: (i, 0)),
        out_shape=jax.ShapeDtypeStruct((M, C), jnp.float32),
    )(gath, xv, fc, r2, sd)


def _comb_pool_body(gath_ref, o_ref):
    o_ref[...] = jnp.max(gath_ref[...], axis=1)


def _comb_pool(gath):
    M, N, C = gath.shape
    BM = _pick_bm(M, N, C)
    return pl.pallas_call(
        _comb_pool_body,
        grid=(M // BM,),
        in_specs=[pl.BlockSpec((BM, N, C), lambda i: (i, 0, 0))],
        out_specs=pl.BlockSpec((BM, C), lambda i: (i, 0)),
        out_shape=jax.ShapeDtypeStruct((M, C), jnp.float32),
    )(gath)


def _nearest_body(src_ref, tgt_ref, o_ref, *, S, V):
    b = pl.program_id(0)
    s = src_ref[0]
    tt = tgt_ref[0]
    inner = (s[:, 0:1] * tt[0:1, :] + s[:, 1:2] * tt[1:2, :]
             + s[:, 2:3] * tt[2:3, :])
    s2 = jnp.sum(s * s, axis=1, keepdims=True)
    t2 = jnp.sum(tt * tt, axis=0, keepdims=True)
    d = s2 + t2 - 2.0 * inner
    m = jnp.min(d, axis=0, keepdims=True)
    iS = lax.broadcasted_iota(jnp.int32, (S, V), 0)
    sel = jnp.where(d == m, iS, S)
    o_ref[0] = jnp.min(sel, axis=0, keepdims=True) + b * S


def _nearest(src, tgt):
    bs, S, _ = src.shape
    V = tgt.shape[1]
    tt = jnp.transpose(tgt, (0, 2, 1))
    return pl.pallas_call(
        functools.partial(_nearest_body, S=S, V=V),
        grid=(bs,),
        in_specs=[pl.BlockSpec((1, S, 3), lambda b: (b, 0, 0)),
                  pl.BlockSpec((1, 3, V), lambda b: (b, 0, 0))],
        out_specs=pl.BlockSpec((1, 1, V), lambda b: (b, 0, 0)),
        out_shape=jax.ShapeDtypeStruct((bs, 1, V), jnp.int32),
    )(src, tt)


def _rowmax_body(x_ref, o_ref):
    o_ref[0] = jnp.max(x_ref[0], axis=0, keepdims=True)


def _rowmax(x):
    bs, V, D = x.shape
    return pl.pallas_call(
        _rowmax_body,
        grid=(bs,),
        in_specs=[pl.BlockSpec((1, V, D), lambda b: (b, 0, 0))],
        out_specs=pl.BlockSpec((1, 1, D), lambda b: (b, 0, 0)),
        out_shape=jax.ShapeDtypeStruct((bs, 1, D), jnp.float32),
    )(x)[:, 0, :]


def _lsm_body(x_ref, o_ref):
    x = x_ref[...]
    m = jnp.max(x, axis=1, keepdims=True)
    sh = x - m
    o_ref[...] = sh - jnp.log(jnp.sum(jnp.exp(sh), axis=1, keepdims=True))


def _lsm(x):
    return pl.pallas_call(
        _lsm_body,
        out_shape=jax.ShapeDtypeStruct(x.shape, jnp.float32),
    )(x)


def _pick_chunk(per_w, D):
    for ch in (128, 96, 80, 64, 48, 40, 32, 24, 16, 8):
        if per_w % ch == 0 and ch * D <= 96 * 1024:
            return ch
    return 8


def _sc_gather(table, idx):
    T, D = table.shape
    (M,) = idx.shape
    per_w = M // _NW
    CH = _pick_chunk(per_w, D)
    nch = per_w // CH
    assert D % 128 == 0, "indirect-stream rows must be tile-aligned"
    mesh = plsc.VectorSubcoreMesh(core_axis_name="c", subcore_axis_name="s",
                                  num_cores=_NC, num_subcores=_NS)

    @functools.partial(
        pl.kernel,
        out_type=jax.ShapeDtypeStruct((M, D), jnp.float32),
        mesh=mesh,
        scratch_types=[pltpu.VMEM((CH,), jnp.int32),
                       pltpu.VMEM((CH, D), jnp.float32),
                       pltpu.SemaphoreType.DMA],
    )
    def gk(table_hbm, idx_hbm, out_hbm, idx_v, rows_v, sem):
        wid = lax.axis_index("s") * _NC + lax.axis_index("c")
        base = wid * per_w

        def body(i, carry):
            off = base + i * CH
            pltpu.sync_copy(idx_hbm.at[pl.ds(off, CH)], idx_v)
            pltpu.async_copy(table_hbm.at[idx_v], rows_v, sem).wait()
            pltpu.sync_copy(rows_v, out_hbm.at[pl.ds(off, CH)])
            return carry

        lax.fori_loop(0, nch, body, 0)

    return gk(table, idx)


def _normdir(sd):
    return sd / jnp.maximum(jnp.linalg.norm(sd, axis=0, keepdims=True), 1e-12)


def _conv_surface(xyz, xv4, r2, sd):
    return _comb_surface(xyz, xv4, r2, _normdir(sd))


def _conv_layer_pair(fmA, fmB, xyzA, xyzB, xv4, niA, niB, r2A, r2B,
                     wA, bA, sdA, wB, bB, sdB, vfp=None):
    C = sdA.shape[1]
    M = fmA.shape[0]
    foA = _mm(fmA, wA, bA)
    foB = _mm(fmB, wB, bB)
    nA = niA.shape[0]
    ni = jnp.concatenate([niA, niB + M])
    if vfp is None:
        tab = jnp.concatenate([foA[:, C:], foB[:, C:]], axis=0)
        g = _sc_gather(tab, ni)
        gathA = g[:nA].reshape(M, nA // M, C)
        gathB = g[nA:].reshape(M, -1, C)
        outA = _comb_layer(gathA, xyzA, xv4, foA[:, :C], r2A, _normdir(sdA))
        outB = _comb_layer(gathB, xyzB, xv4, foB[:, :C], r2B, _normdir(sdB))
        return outA, outB, xyzA, xyzB
    tab = jnp.concatenate(
        [jnp.concatenate([vfp, foA[:, C:]], axis=1),
         jnp.concatenate([vfp, foB[:, C:]], axis=1)], axis=0)
    g = _sc_gather(tab, ni)
    gathA = g[:nA].reshape(M, nA // M, 128 + C)
    gathB = g[nA:].reshape(M, -1, 128 + C)
    outA = _comb_layer_packed(gathA, xv4, foA[:, :C], r2A, _normdir(sdA), 128)
    outB = _comb_layer_packed(gathB, xv4, foB[:, :C], r2B, _normdir(sdB), 128)
    return outA, outB, gathA[:, :, :4], gathB[:, :, :4]


def kernel(vertices, onehot, params):
    p = params
    bs = vertices.shape[0]
    V = vertices.shape[2]
    verts = jnp.transpose(vertices, (0, 2, 1))
    M1 = bs * V
    vf = verts.reshape(M1, 3)

    idx51, d51 = _topk(verts, 51, 256)
    ni50 = idx51[:, :, 1:51].reshape(-1)
    r2_50 = d51[:, :, 1:51].reshape(M1, 50)
    ni10 = idx51[:, :, 1:11].reshape(-1)
    r2_10 = d51[:, :, 1:11].reshape(M1, 10)
    ni4 = idx51[:, :, 1:5].reshape(-1)

    vf4 = jnp.pad(vf, ((0, 0), (0, 1)))
    vfp1 = jnp.pad(vf, ((0, 0), (0, 125)))
    xyzb = _sc_gather(vfp1, jnp.concatenate([ni50, ni10]))
    xyz50 = xyzb[:M1 * 50].reshape(M1, 50, 128)[:, :, :4]
    xyz10 = xyzb[M1 * 50:].reshape(M1, 10, 128)[:, :, :4]

    fm0 = _bn(_conv_surface(xyz50, vf4, r2_50, p["conv0_dir"]),
              p["bn0_g"], p["bn0_b"])
    fm0l = _bn(_conv_surface(xyz10, vf4, r2_10, p["conv0l_dir"]),
               p["bn0l_g"], p["bn0l_b"])
    c1, c1l, _, _ = _conv_layer_pair(
        fm0, fm0l, xyz50, xyz10, vf4, ni50, ni10, r2_50, r2_10,
        p["conv1_w"], p["conv1_b"], p["conv1_dir"],
        p["conv1l_w"], p["conv1l_b"], p["conv1l_dir"])
    fm1 = _bn(c1, p["bn1_g"], p["bn1_b"])
    fm1l = _bn(c1l, p["bn1l_g"], p["bn1l_b"])
    fm1t = _mm(jnp.concatenate([fm1, fm1l], axis=1),
               p["down0_w"], p["down0_b"])

    V1 = V // 4
    perm1 = jax.random.permutation(jax.random.key(1), V)[:V1]
    pooled = _comb_pool(_sc_gather(fm1t, ni4).reshape(M1, 4, 128))
    fp1 = pooled.reshape(bs, V, 128)[:, perm1].reshape(bs * V1, 128)
    vp1 = verts[:, perm1]
    v1f = vp1.reshape(bs * V1, 3)
    M2 = bs * V1

    idx51_2, d51_2 = _topk(vp1, 51, V1)
    n2_50 = idx51_2[:, :, 1:51].reshape(-1)
    q2_50 = d51_2[:, :, 1:51].reshape(M2, 50)
    n2_10 = idx51_2[:, :, 1:11].reshape(-1)
    q2_10 = d51_2[:, :, 1:11].reshape(M2, 10)
    n2_4 = idx51_2[:, :, 1:5].reshape(-1)

    v1f4 = jnp.pad(v1f, ((0, 0), (0, 1)))
    vfp2 = jnp.pad(v1f, ((0, 0), (0, 125)))
    c2, c2l, xyz2_50, xyz2_10 = _conv_layer_pair(
        fp1, fp1, None, None, v1f4, n2_50, n2_10, q2_50, q2_10,
        p["conv2_w"], p["conv2_b"], p["conv2_dir"],
        p["conv2l_w"], p["conv2l_b"], p["conv2l_dir"], vfp=vfp2)
    fm2 = _bn(c2, p["bn2_g"], p["bn2_b"])
    fm2l = _bn(c2l, p["bn2l_g"], p["bn2l_b"])
    c3, c3l, _, _ = _conv_layer_pair(
        fm2, fm2l, xyz2_50, xyz2_10, v1f4, n2_50, n2_10, q2_50, q2_10,
        p["conv3_w"], p["conv3_b"], p["conv3_dir"],
        p["conv3l_w"], p["conv3l_b"], p["conv3l_dir"])
    fm3 = _bn(c3, p["bn3_g"], p["bn3_b"])
    fm3l = _bn(c3l, p["bn3l_g"], p["bn3l_b"])
    fm3t = _mm(jnp.concatenate([fm3, fm3l], axis=1),
               p["down1_w"], p["down1_b"])

    V2 = V1 // 4
    perm2 = jax.random.permutation(jax.random.key(2), V1)[:V2]
    pooled2 = _comb_pool(_sc_gather(fm3t, n2_4).reshape(M2, 4, 256))
    fp2 = pooled2.reshape(bs, V1, 256)[:, perm2].reshape(bs * V2, 256)
    vp2 = vp1[:, perm2]
    v2f = vp2.reshape(bs * V2, 3)
    M3 = bs * V2

    idx51_3, d51_3 = _topk(vp2, 51, V2)
    n3_50 = idx51_3[:, :, 1:51].reshape(-1)
    q3_50 = d51_3[:, :, 1:51].reshape(M3, 50)
    n3_10 = idx51_3[:, :, 1:11].reshape(-1)
    q3_10 = d51_3[:, :, 1:11].reshape(M3, 10)

    v2f4 = jnp.pad(v2f, ((0, 0), (0, 1)))
    vfp3 = jnp.pad(v2f, ((0, 0), (0, 125)))
    fm4, fm4l, _, _ = _conv_layer_pair(
        fp2, fp2, None, None, v2f4, n3_50, n3_10, q3_50, q3_10,
        p["conv4_w"], p["conv4_b"], p["conv4_dir"],
        p["conv4l_w"], p["conv4l_b"], p["conv4l_dir"], vfp=vfp3)
    fm4t = _mm(jnp.concatenate([fm4, fm4l], axis=1),
               p["down2_w"], p["down2_b"])

    fglob = _rowmax(fm4t.reshape(bs, V2, 512))

    np1 = _nearest(vp1, verts).reshape(-1)
    np2 = _nearest(vp2, verts).reshape(-1)
    tabn = jnp.concatenate([jnp.pad(fm3t, ((0, 0), (0, 256))), fm4t], axis=0)
    gn = _sc_gather(tabn, jnp.concatenate([np1, np2 + M2]))
    fm3f = gn[:M1, :256]
    fm4f = gn[M1:]
    fg = jnp.broadcast_to(fglob[:, None, :], (bs, V, 512)).reshape(M1, 512)
    oh = jnp.broadcast_to(onehot[:, None, :],
                          (bs, V, onehot.shape[1])).reshape(M1, -1)
    fuse = jnp.concatenate([fm1t, fm3f, fm4f, fg, oh], axis=1)

    x = _mm(fuse, p["h1_w"].T, p["h1_b"], act="relu")
    x = _mm(x, p["h2_w"].T, p["h2_b"], act="relu")
    x = _mm(x, p["h3_w"].T, p["h3_b"])
    return _lsm(x).reshape(bs, V, 50)


# speedup vs baseline: 1.0708x; 1.0708x over previous
import functools

import jax
import jax.numpy as jnp
from jax import lax
from jax.experimental import pallas as pl
from jax.experimental.pallas import tpu as pltpu
from jax.experimental.pallas import tpu_sc as plsc

_NC, _NS = 2, 16
_NW = _NC * _NS


def _topk_body(vr_ref, vt_ref, idx_ref, d_ref, *, K1, V):
    b = pl.program_id(0)
    vr = vr_ref[0]
    vt = vt_ref[0]
    R = vr.shape[0]
    inner = (vr[:, 0:1] * vt[0:1, :] + vr[:, 1:2] * vt[1:2, :]
             + vr[:, 2:3] * vt[2:3, :])
    qc = jnp.sum(vt * vt, axis=0, keepdims=True)
    qr = jnp.sum(vr * vr, axis=1, keepdims=True)
    neg = 2.0 * inner - qc - qr
    iota = lax.broadcasted_iota(jnp.int32, (R, V), 1)
    iok = lax.broadcasted_iota(jnp.int32, (R, K1), 1)
    negbig = jnp.float32(-3e38)

    def step(t, carry):
        neg, ai, ad = carry
        m = jnp.max(neg, axis=1, keepdims=True)
        sel = jnp.where(neg == m, iota, V)
        j = jnp.min(sel, axis=1, keepdims=True)
        ai = jnp.where(iok == t, j, ai)
        ad = jnp.where(iok == t, -m, ad)
        neg = jnp.where(iota == j, negbig, neg)
        return neg, ai, ad

    ai0 = jnp.zeros((R, K1), jnp.int32)
    ad0 = jnp.zeros((R, K1), jnp.float32)
    _, ai, ad = lax.fori_loop(0, K1, step, (neg, ai0, ad0))
    idx_ref[0] = ai + b * V
    d_ref[0] = ad


def _topk(verts, K1, R):
    bs, V, _ = verts.shape
    vt = jnp.transpose(verts, (0, 2, 1))
    return pl.pallas_call(
        functools.partial(_topk_body, K1=K1, V=V),
        grid=(bs, V // R),
        in_specs=[pl.BlockSpec((1, R, 3), lambda b, i: (b, i, 0)),
                  pl.BlockSpec((1, 3, V), lambda b, i: (b, 0, 0))],
        out_specs=[pl.BlockSpec((1, R, K1), lambda b, i: (b, i, 0)),
                   pl.BlockSpec((1, R, K1), lambda b, i: (b, i, 0))],
        out_shape=[jax.ShapeDtypeStruct((bs, V, K1), jnp.int32),
                   jax.ShapeDtypeStruct((bs, V, K1), jnp.float32)],
    )(verts, vt)


def _mm_body(x_ref, w_ref, b_ref, o_ref, *, act, K):
    x = x_ref[...]
    w = w_ref[...]
    if K <= 4:
        acc = x[:, 0:1] * w[0:1, :]
        for k in range(1, K):
            acc = acc + x[:, k:k + 1] * w[k:k + 1, :]
    else:
        acc = jnp.dot(x, w, preferred_element_type=jnp.float32)
    acc = acc + b_ref[...]
    if act == "relu":
        acc = jnp.maximum(acc, 0.0)
    o_ref[...] = acc


def _mm(x, w, b=None, act=None):
    M, K = x.shape
    N = w.shape[1]
    if b is None:
        b = jnp.zeros((N,), jnp.float32)
    BM = min(1024, M)
    return pl.pallas_call(
        functools.partial(_mm_body, act=act, K=K),
        grid=(M // BM,),
        in_specs=[pl.BlockSpec((BM, K), lambda i: (i, 0)),
                  pl.BlockSpec((K, N), lambda i: (0, 0)),
                  pl.BlockSpec((1, N), lambda i: (0, 0))],
        out_specs=pl.BlockSpec((BM, N), lambda i: (i, 0)),
        out_shape=jax.ShapeDtypeStruct((M, N), jnp.float32),
    )(x, w, b[None, :])


def _bn_body(x_ref, g_ref, b_ref, o_ref, *, relu):
    x = x_ref[...]
    m = jnp.mean(x, axis=0, keepdims=True)
    v = jnp.mean((x - m) ** 2, axis=0, keepdims=True)
    y = g_ref[...] * (x - m) / jnp.sqrt(v + 1e-5) + b_ref[...]
    if relu:
        y = jnp.maximum(y, 0.0)
    o_ref[...] = y


def _bn(x, g, b, relu=True):
    M, C = x.shape
    return pl.pallas_call(
        functools.partial(_bn_body, relu=relu),
        out_shape=jax.ShapeDtypeStruct((M, C), jnp.float32),
    )(x, g[None, :], b[None, :])


def _pick_bm(M, N, D):
    for bm in (1024, 512, 256, 128, 64, 32):
        if M % bm == 0 and bm * N * D * 4 <= 6 * 2**20:
            return bm
    return 32


def _rinv(r2):
    return 1.0 / jnp.maximum(jnp.sqrt(jnp.maximum(r2, 0.0)), 1e-12)


def _theta(xyz, xv, r2, sd4):
    BM, N = r2.shape
    C = sd4.shape[1]
    ri = _rinv(r2)[:, :, None]
    diff = xyz[:, :, :4] - xv[:, None, :]
    scaled = diff * ri
    dot = jnp.dot(scaled.reshape(BM * N, 4), sd4,
                  preferred_element_type=jnp.float32).reshape(BM, N, C)
    return jnp.maximum(dot, 0.0)


def _comb_surface_body(xyz_ref, xv_ref, r2_ref, sd_ref, o_ref):
    th = _theta(xyz_ref[...], xv_ref[...], r2_ref[...], sd_ref[...])
    o_ref[...] = jnp.max(th, axis=1)


def _comb_surface(xyz, xv, r2, sd):
    M, N, W = xyz.shape
    C = sd.shape[1]
    BM = _pick_bm(M, N, 128 + C)
    return pl.pallas_call(
        _comb_surface_body,
        grid=(M // BM,),
        in_specs=[pl.BlockSpec((BM, N, W), lambda i: (i, 0, 0)),
                  pl.BlockSpec((BM, 4), lambda i: (i, 0)),
                  pl.BlockSpec((BM, N), lambda i: (i, 0)),
                  pl.BlockSpec((4, C), lambda i: (0, 0))],
        out_specs=pl.BlockSpec((BM, C), lambda i: (i, 0)),
        out_shape=jax.ShapeDtypeStruct((M, C), jnp.float32),
    )(xyz, xv, r2, jnp.pad(sd, ((0, 1), (0, 0))))


def _comb_layer_body(fs_ref, xyz_ref, xv_ref, fc_ref, r2_ref, sd_ref, o_ref):
    th = _theta(xyz_ref[...], xv_ref[...], r2_ref[...], sd_ref[...])
    o_ref[...] = fc_ref[...] + jnp.max(th * fs_ref[...], axis=1)


def _comb_layer(fs, xyz, xv, fc, r2, sd):
    M, N, C = fs.shape
    W = xyz.shape[2]
    BM = _pick_bm(M, N, C + 256)
    return pl.pallas_call(
        _comb_layer_body,
        grid=(M // BM,),
        in_specs=[pl.BlockSpec((BM, N, C), lambda i: (i, 0, 0)),
                  pl.BlockSpec((BM, N, W), lambda i: (i, 0, 0)),
                  pl.BlockSpec((BM, 4), lambda i: (i, 0)),
                  pl.BlockSpec((BM, C), lambda i: (i, 0)),
                  pl.BlockSpec((BM, N), lambda i: (i, 0)),
                  pl.BlockSpec((4, C), lambda i: (0, 0))],
        out_specs=pl.BlockSpec((BM, C), lambda i: (i, 0)),
        out_shape=jax.ShapeDtypeStruct((M, C), jnp.float32),
    )(fs, xyz, xv, fc, r2, jnp.pad(sd, ((0, 1), (0, 0))))


def _comb_layer_packed_body(g_ref, xv_ref, fc_ref, r2_ref, sd_ref, o_ref, *,
                            fsoff):
    x = g_ref[...]
    th = _theta(x, xv_ref[...], r2_ref[...], sd_ref[...])
    o_ref[...] = fc_ref[...] + jnp.max(th * x[:, :, fsoff:], axis=1)


def _comb_layer_packed(gath, xv, fc, r2, sd, fsoff):
    M, N, D = gath.shape
    C = D - fsoff
    BM = _pick_bm(M, N, D + C)
    return pl.pallas_call(
        functools.partial(_comb_layer_packed_body, fsoff=fsoff),
        grid=(M // BM,),
        in_specs=[pl.BlockSpec((BM, N, D), lambda i: (i, 0, 0)),
                  pl.BlockSpec((BM, 4), lambda i: (i, 0)),
                  pl.BlockSpec((BM, C), lambda i: (i, 0)),
                  pl.BlockSpec((BM, N), lambda i: (i, 0)),
                  pl.BlockSpec((4, C), lambda i: (0, 0))],
        out_specs=pl.BlockSpec((BM, C), lambda i: (i, 0)),
        out_shape=jax.ShapeDtypeStruct((M, C), jnp.float32),
    )(gath, xv, fc, r2, jnp.pad(sd, ((0, 1), (0, 0))))


def _comb_pool_body(gath_ref, o_ref):
    o_ref[...] = jnp.max(gath_ref[...], axis=1)


def _comb_pool(gath):
    M, N, C = gath.shape
    BM = _pick_bm(M, N, C)
    return pl.pallas_call(
        _comb_pool_body,
        grid=(M // BM,),
        in_specs=[pl.BlockSpec((BM, N, C), lambda i: (i, 0, 0))],
        out_specs=pl.BlockSpec((BM, C), lambda i: (i, 0)),
        out_shape=jax.ShapeDtypeStruct((M, C), jnp.float32),
    )(gath)


def _nearest_body(src_ref, tgt_ref, o_ref, *, S, V):
    b = pl.program_id(0)
    s = src_ref[0]
    tt = tgt_ref[0]
    inner = (s[:, 0:1] * tt[0:1, :] + s[:, 1:2] * tt[1:2, :]
             + s[:, 2:3] * tt[2:3, :])
    s2 = jnp.sum(s * s, axis=1, keepdims=True)
    t2 = jnp.sum(tt * tt, axis=0, keepdims=True)
    d = s2 + t2 - 2.0 * inner
    m = jnp.min(d, axis=0, keepdims=True)
    iS = lax.broadcasted_iota(jnp.int32, (S, V), 0)
    sel = jnp.where(d == m, iS, S)
    o_ref[0] = jnp.min(sel, axis=0, keepdims=True) + b * S


def _nearest(src, tgt):
    bs, S, _ = src.shape
    V = tgt.shape[1]
    tt = jnp.transpose(tgt, (0, 2, 1))
    return pl.pallas_call(
        functools.partial(_nearest_body, S=S, V=V),
        grid=(bs,),
        in_specs=[pl.BlockSpec((1, S, 3), lambda b: (b, 0, 0)),
                  pl.BlockSpec((1, 3, V), lambda b: (b, 0, 0))],
        out_specs=pl.BlockSpec((1, 1, V), lambda b: (b, 0, 0)),
        out_shape=jax.ShapeDtypeStruct((bs, 1, V), jnp.int32),
    )(src, tt)


def _rowmax_body(x_ref, o_ref):
    o_ref[0] = jnp.max(x_ref[0], axis=0, keepdims=True)


def _rowmax(x):
    bs, V, D = x.shape
    return pl.pallas_call(
        _rowmax_body,
        grid=(bs,),
        in_specs=[pl.BlockSpec((1, V, D), lambda b: (b, 0, 0))],
        out_specs=pl.BlockSpec((1, 1, D), lambda b: (b, 0, 0)),
        out_shape=jax.ShapeDtypeStruct((bs, 1, D), jnp.float32),
    )(x)[:, 0, :]


def _lsm_body(x_ref, o_ref):
    x = x_ref[...]
    m = jnp.max(x, axis=1, keepdims=True)
    sh = x - m
    o_ref[...] = sh - jnp.log(jnp.sum(jnp.exp(sh), axis=1, keepdims=True))


def _lsm(x):
    return pl.pallas_call(
        _lsm_body,
        out_shape=jax.ShapeDtypeStruct(x.shape, jnp.float32),
    )(x)


def _pick_chunk(per_w, D):
    for ch in (128, 96, 80, 64, 48, 40, 32, 24, 16, 8):
        if per_w % ch == 0 and ch * D <= 96 * 1024:
            return ch
    return 8


def _sc_gather(table, idx):
    T, D = table.shape
    (M,) = idx.shape
    per_w = M // _NW
    CH = _pick_chunk(per_w, D)
    nch = per_w // CH
    assert D % 128 == 0, "indirect-stream rows must be tile-aligned"
    mesh = plsc.VectorSubcoreMesh(core_axis_name="c", subcore_axis_name="s",
                                  num_cores=_NC, num_subcores=_NS)

    @functools.partial(
        pl.kernel,
        out_type=jax.ShapeDtypeStruct((M, D), jnp.float32),
        mesh=mesh,
        scratch_types=[pltpu.VMEM((CH,), jnp.int32),
                       pltpu.VMEM((CH, D), jnp.float32),
                       pltpu.SemaphoreType.DMA],
    )
    def gk(table_hbm, idx_hbm, out_hbm, idx_v, rows_v, sem):
        wid = lax.axis_index("s") * _NC + lax.axis_index("c")
        base = wid * per_w

        def body(i, carry):
            off = base + i * CH
            pltpu.sync_copy(idx_hbm.at[pl.ds(off, CH)], idx_v)
            pltpu.async_copy(table_hbm.at[idx_v], rows_v, sem).wait()
            pltpu.sync_copy(rows_v, out_hbm.at[pl.ds(off, CH)])
            return carry

        lax.fori_loop(0, nch, body, 0)

    return gk(table, idx)


def _normdir(sd):
    return sd / jnp.maximum(jnp.linalg.norm(sd, axis=0, keepdims=True), 1e-12)


def _conv_surface(xyz, xv4, r2, sd):
    return _comb_surface(xyz, xv4, r2, _normdir(sd))


def _conv_layer_pair(fmA, fmB, xyzA, xyzB, xv4, niA, niB, r2A, r2B,
                     wA, bA, sdA, wB, bB, sdB, vfp=None):
    C = sdA.shape[1]
    M = fmA.shape[0]
    foA = _mm(fmA, wA, bA)
    foB = _mm(fmB, wB, bB)
    nA = niA.shape[0]
    ni = jnp.concatenate([niA, niB + M])
    if vfp is None:
        tab = jnp.concatenate([foA[:, C:], foB[:, C:]], axis=0)
        g = _sc_gather(tab, ni)
        gathA = g[:nA].reshape(M, nA // M, C)
        gathB = g[nA:].reshape(M, -1, C)
        outA = _comb_layer(gathA, xyzA, xv4, foA[:, :C], r2A, _normdir(sdA))
        outB = _comb_layer(gathB, xyzB, xv4, foB[:, :C], r2B, _normdir(sdB))
        return outA, outB, xyzA, xyzB
    tab = jnp.concatenate(
        [jnp.concatenate([vfp, foA[:, C:]], axis=1),
         jnp.concatenate([vfp, foB[:, C:]], axis=1)], axis=0)
    g = _sc_gather(tab, ni)
    gathA = g[:nA].reshape(M, nA // M, 128 + C)
    gathB = g[nA:].reshape(M, -1, 128 + C)
    outA = _comb_layer_packed(gathA, xv4, foA[:, :C], r2A, _normdir(sdA), 128)
    outB = _comb_layer_packed(gathB, xv4, foB[:, :C], r2B, _normdir(sdB), 128)
    return outA, outB, gathA[:, :, :4], gathB[:, :, :4]


def kernel(vertices, onehot, params):
    p = params
    bs = vertices.shape[0]
    V = vertices.shape[2]
    verts = jnp.transpose(vertices, (0, 2, 1))
    M1 = bs * V
    vf = verts.reshape(M1, 3)

    idx51, d51 = _topk(verts, 51, 256)
    ni50 = idx51[:, :, 1:51].reshape(-1)
    r2_50 = d51[:, :, 1:51].reshape(M1, 50)
    ni10 = idx51[:, :, 1:11].reshape(-1)
    r2_10 = d51[:, :, 1:11].reshape(M1, 10)
    ni4 = idx51[:, :, 1:5].reshape(-1)

    vf4 = jnp.pad(vf, ((0, 0), (0, 1)))
    vfp1 = jnp.pad(vf, ((0, 0), (0, 125)))
    xyzb = _sc_gather(vfp1, ni50)
    xyz50 = xyzb.reshape(M1, 50, 128)[:, :, :4]
    xyz10 = xyz50[:, :10]

    fm0 = _bn(_conv_surface(xyz50, vf4, r2_50, p["conv0_dir"]),
              p["bn0_g"], p["bn0_b"])
    fm0l = _bn(_conv_surface(xyz10, vf4, r2_10, p["conv0l_dir"]),
               p["bn0l_g"], p["bn0l_b"])
    c1, c1l, _, _ = _conv_layer_pair(
        fm0, fm0l, xyz50, xyz10, vf4, ni50, ni10, r2_50, r2_10,
        p["conv1_w"], p["conv1_b"], p["conv1_dir"],
        p["conv1l_w"], p["conv1l_b"], p["conv1l_dir"])
    fm1 = _bn(c1, p["bn1_g"], p["bn1_b"])
    fm1l = _bn(c1l, p["bn1l_g"], p["bn1l_b"])
    fm1t = _mm(jnp.concatenate([fm1, fm1l], axis=1),
               p["down0_w"], p["down0_b"])

    V1 = V // 4
    perm1 = jax.random.permutation(jax.random.key(1), V)[:V1]
    pooled = _comb_pool(_sc_gather(fm1t, ni4).reshape(M1, 4, 128))
    fp1 = pooled.reshape(bs, V, 128)[:, perm1].reshape(bs * V1, 128)
    vp1 = verts[:, perm1]
    v1f = vp1.reshape(bs * V1, 3)
    M2 = bs * V1

    idx51_2, d51_2 = _topk(vp1, 51, V1)
    n2_50 = idx51_2[:, :, 1:51].reshape(-1)
    q2_50 = d51_2[:, :, 1:51].reshape(M2, 50)
    n2_10 = idx51_2[:, :, 1:11].reshape(-1)
    q2_10 = d51_2[:, :, 1:11].reshape(M2, 10)
    n2_4 = idx51_2[:, :, 1:5].reshape(-1)

    v1f4 = jnp.pad(v1f, ((0, 0), (0, 1)))
    vfp2 = jnp.pad(v1f, ((0, 0), (0, 125)))
    c2, c2l, xyz2_50, xyz2_10 = _conv_layer_pair(
        fp1, fp1, None, None, v1f4, n2_50, n2_10, q2_50, q2_10,
        p["conv2_w"], p["conv2_b"], p["conv2_dir"],
        p["conv2l_w"], p["conv2l_b"], p["conv2l_dir"], vfp=vfp2)
    fm2 = _bn(c2, p["bn2_g"], p["bn2_b"])
    fm2l = _bn(c2l, p["bn2l_g"], p["bn2l_b"])
    c3, c3l, _, _ = _conv_layer_pair(
        fm2, fm2l, xyz2_50, xyz2_10, v1f4, n2_50, n2_10, q2_50, q2_10,
        p["conv3_w"], p["conv3_b"], p["conv3_dir"],
        p["conv3l_w"], p["conv3l_b"], p["conv3l_dir"])
    fm3 = _bn(c3, p["bn3_g"], p["bn3_b"])
    fm3l = _bn(c3l, p["bn3l_g"], p["bn3l_b"])
    fm3t = _mm(jnp.concatenate([fm3, fm3l], axis=1),
               p["down1_w"], p["down1_b"])

    V2 = V1 // 4
    perm2 = jax.random.permutation(jax.random.key(2), V1)[:V2]
    pooled2 = _comb_pool(_sc_gather(fm3t, n2_4).reshape(M2, 4, 256))
    fp2 = pooled2.reshape(bs, V1, 256)[:, perm2].reshape(bs * V2, 256)
    vp2 = vp1[:, perm2]
    v2f = vp2.reshape(bs * V2, 3)
    M3 = bs * V2

    idx51_3, d51_3 = _topk(vp2, 51, V2)
    n3_50 = idx51_3[:, :, 1:51].reshape(-1)
    q3_50 = d51_3[:, :, 1:51].reshape(M3, 50)
    n3_10 = idx51_3[:, :, 1:11].reshape(-1)
    q3_10 = d51_3[:, :, 1:11].reshape(M3, 10)

    v2f4 = jnp.pad(v2f, ((0, 0), (0, 1)))
    vfp3 = jnp.pad(v2f, ((0, 0), (0, 125)))
    fm4, fm4l, _, _ = _conv_layer_pair(
        fp2, fp2, None, None, v2f4, n3_50, n3_10, q3_50, q3_10,
        p["conv4_w"], p["conv4_b"], p["conv4_dir"],
        p["conv4l_w"], p["conv4l_b"], p["conv4l_dir"], vfp=vfp3)
    fm4t = _mm(jnp.concatenate([fm4, fm4l], axis=1),
               p["down2_w"], p["down2_b"])

    fglob = _rowmax(fm4t.reshape(bs, V2, 512))

    np1 = _nearest(vp1, verts).reshape(-1)
    np2 = _nearest(vp2, verts).reshape(-1)
    tabn = jnp.concatenate([jnp.pad(fm3t, ((0, 0), (0, 256))), fm4t], axis=0)
    gn = _sc_gather(tabn, jnp.concatenate([np1, np2 + M2]))
    fm3f = gn[:M1, :256]
    fm4f = gn[M1:]
    fg = jnp.broadcast_to(fglob[:, None, :], (bs, V, 512)).reshape(M1, 512)
    oh = jnp.broadcast_to(onehot[:, None, :],
                          (bs, V, onehot.shape[1])).reshape(M1, -1)
    fuse = jnp.concatenate([fm1t, fm3f, fm4f, fg, oh], axis=1)

    x = _mm(fuse, p["h1_w"].T, p["h1_b"], act="relu")
    x = _mm(x, p["h2_w"].T, p["h2_b"], act="relu")
    x = _mm(x, p["h3_w"].T, p["h3_b"])
    return _lsm(x).reshape(bs, V, 50)


# speedup vs baseline: 1.1789x; 1.1009x over previous
import functools

import jax
import jax.numpy as jnp
from jax import lax
from jax.experimental import pallas as pl
from jax.experimental.pallas import tpu as pltpu
from jax.experimental.pallas import tpu_sc as plsc

_NC, _NS = 2, 16
_NW = _NC * _NS


def _topk_body(vr_ref, vt_ref, idx_ref, d_ref, *, K1, V):
    b = pl.program_id(0)
    vr = vr_ref[0]
    vt = vt_ref[0]
    R = vr.shape[0]
    inner = (vr[:, 0:1] * vt[0:1, :] + vr[:, 1:2] * vt[1:2, :]
             + vr[:, 2:3] * vt[2:3, :])
    qc = jnp.sum(vt * vt, axis=0, keepdims=True)
    qr = jnp.sum(vr * vr, axis=1, keepdims=True)
    neg = 2.0 * inner - qc - qr
    bits = lax.bitcast_convert_type(neg, jnp.int32)
    srt = jnp.where(bits >= 0, bits, bits ^ jnp.int32(0x7FFFFFFF))
    iota = lax.broadcasted_iota(jnp.int32, (R, V), 1)
    keys = (srt & jnp.int32(~0x7FF)) | (jnp.int32(2047) - iota)
    iok = lax.broadcasted_iota(jnp.int32, (R, K1), 1)
    kmin = jnp.int32(-0x80000000)

    def step(t, carry):
        keys, ak = carry
        m = jnp.max(keys, axis=1, keepdims=True)
        ak = jnp.where(iok == t, m, ak)
        keys = jnp.where(keys == m, kmin, keys)
        return keys, ak

    ak0 = jnp.zeros((R, K1), jnp.int32)
    _, ak = lax.fori_loop(0, K1, step, (keys, ak0))
    idx_ref[0] = (jnp.int32(2047) - (ak & jnp.int32(0x7FF))) + b * V
    vb = ak & jnp.int32(~0x7FF)
    vb = jnp.where(vb >= 0, vb, vb ^ jnp.int32(0x7FFFFFFF))
    d_ref[0] = -lax.bitcast_convert_type(vb, jnp.float32)


def _topk(verts, K1, R):
    bs, V, _ = verts.shape
    vt = jnp.transpose(verts, (0, 2, 1))
    return pl.pallas_call(
        functools.partial(_topk_body, K1=K1, V=V),
        grid=(bs, V // R),
        in_specs=[pl.BlockSpec((1, R, 3), lambda b, i: (b, i, 0)),
                  pl.BlockSpec((1, 3, V), lambda b, i: (b, 0, 0))],
        out_specs=[pl.BlockSpec((1, R, K1), lambda b, i: (b, i, 0)),
                   pl.BlockSpec((1, R, K1), lambda b, i: (b, i, 0))],
        out_shape=[jax.ShapeDtypeStruct((bs, V, K1), jnp.int32),
                   jax.ShapeDtypeStruct((bs, V, K1), jnp.float32)],
    )(verts, vt)


def _mm_body(x_ref, w_ref, b_ref, o_ref, *, act, K):
    x = x_ref[...]
    w = w_ref[...]
    if K <= 4:
        acc = x[:, 0:1] * w[0:1, :]
        for k in range(1, K):
            acc = acc + x[:, k:k + 1] * w[k:k + 1, :]
    else:
        acc = jnp.dot(x, w, preferred_element_type=jnp.float32)
    acc = acc + b_ref[...]
    if act == "relu":
        acc = jnp.maximum(acc, 0.0)
    o_ref[...] = acc


def _mm(x, w, b=None, act=None):
    M, K = x.shape
    N = w.shape[1]
    if b is None:
        b = jnp.zeros((N,), jnp.float32)
    BM = min(1024, M)
    return pl.pallas_call(
        functools.partial(_mm_body, act=act, K=K),
        grid=(M // BM,),
        in_specs=[pl.BlockSpec((BM, K), lambda i: (i, 0)),
                  pl.BlockSpec((K, N), lambda i: (0, 0)),
                  pl.BlockSpec((1, N), lambda i: (0, 0))],
        out_specs=pl.BlockSpec((BM, N), lambda i: (i, 0)),
        out_shape=jax.ShapeDtypeStruct((M, N), jnp.float32),
    )(x, w, b[None, :])


def _bn_body(x_ref, g_ref, b_ref, o_ref, *, relu):
    x = x_ref[...]
    m = jnp.mean(x, axis=0, keepdims=True)
    v = jnp.mean((x - m) ** 2, axis=0, keepdims=True)
    y = g_ref[...] * (x - m) / jnp.sqrt(v + 1e-5) + b_ref[...]
    if relu:
        y = jnp.maximum(y, 0.0)
    o_ref[...] = y


def _bn(x, g, b, relu=True):
    M, C = x.shape
    return pl.pallas_call(
        functools.partial(_bn_body, relu=relu),
        out_shape=jax.ShapeDtypeStruct((M, C), jnp.float32),
    )(x, g[None, :], b[None, :])


def _pick_bm(M, N, D):
    for bm in (1024, 512, 256, 128, 64, 32):
        if M % bm == 0 and bm * N * D * 4 <= 6 * 2**20:
            return bm
    return 32


def _rinv(r2):
    return 1.0 / jnp.maximum(jnp.sqrt(jnp.maximum(r2, 0.0)), 1e-12)


def _theta(xyz, xv, r2, sd4):
    BM, N = r2.shape
    C = sd4.shape[1]
    ri = _rinv(r2)[:, :, None]
    diff = xyz[:, :, :4] - xv[:, None, :]
    scaled = diff * ri
    dot = jnp.dot(scaled.reshape(BM * N, 4), sd4,
                  preferred_element_type=jnp.float32).reshape(BM, N, C)
    return jnp.maximum(dot, 0.0)


def _comb_surface_body(xyz_ref, xv_ref, r2_ref, sd_ref, o_ref):
    th = _theta(xyz_ref[...], xv_ref[...], r2_ref[...], sd_ref[...])
    o_ref[...] = jnp.max(th, axis=1)


def _comb_surface(xyz, xv, r2, sd):
    M, N, W = xyz.shape
    C = sd.shape[1]
    BM = _pick_bm(M, N, 128 + C)
    return pl.pallas_call(
        _comb_surface_body,
        grid=(M // BM,),
        in_specs=[pl.BlockSpec((BM, N, W), lambda i: (i, 0, 0)),
                  pl.BlockSpec((BM, 4), lambda i: (i, 0)),
                  pl.BlockSpec((BM, N), lambda i: (i, 0)),
                  pl.BlockSpec((4, C), lambda i: (0, 0))],
        out_specs=pl.BlockSpec((BM, C), lambda i: (i, 0)),
        out_shape=jax.ShapeDtypeStruct((M, C), jnp.float32),
    )(xyz, xv, r2, jnp.pad(sd, ((0, 1), (0, 0))))


def _comb_layer_body(fs_ref, xyz_ref, xv_ref, fc_ref, r2_ref, sd_ref, o_ref):
    th = _theta(xyz_ref[...], xv_ref[...], r2_ref[...], sd_ref[...])
    o_ref[...] = fc_ref[...] + jnp.max(th * fs_ref[...], axis=1)


def _comb_layer(fs, xyz, xv, fc, r2, sd):
    M, N, C = fs.shape
    W = xyz.shape[2]
    BM = _pick_bm(M, N, C + 256)
    return pl.pallas_call(
        _comb_layer_body,
        grid=(M // BM,),
        in_specs=[pl.BlockSpec((BM, N, C), lambda i: (i, 0, 0)),
                  pl.BlockSpec((BM, N, W), lambda i: (i, 0, 0)),
                  pl.BlockSpec((BM, 4), lambda i: (i, 0)),
                  pl.BlockSpec((BM, C), lambda i: (i, 0)),
                  pl.BlockSpec((BM, N), lambda i: (i, 0)),
                  pl.BlockSpec((4, C), lambda i: (0, 0))],
        out_specs=pl.BlockSpec((BM, C), lambda i: (i, 0)),
        out_shape=jax.ShapeDtypeStruct((M, C), jnp.float32),
    )(fs, xyz, xv, fc, r2, jnp.pad(sd, ((0, 1), (0, 0))))


def _comb_layer_packed_body(g_ref, xv_ref, fc_ref, r2_ref, sd_ref, o_ref, *,
                            fsoff):
    x = g_ref[...]
    th = _theta(x, xv_ref[...], r2_ref[...], sd_ref[...])
    o_ref[...] = fc_ref[...] + jnp.max(th * x[:, :, fsoff:], axis=1)


def _comb_layer_packed(gath, xv, fc, r2, sd, fsoff):
    M, N, D = gath.shape
    C = D - fsoff
    BM = _pick_bm(M, N, D + C)
    return pl.pallas_call(
        functools.partial(_comb_layer_packed_body, fsoff=fsoff),
        grid=(M // BM,),
        in_specs=[pl.BlockSpec((BM, N, D), lambda i: (i, 0, 0)),
                  pl.BlockSpec((BM, 4), lambda i: (i, 0)),
                  pl.BlockSpec((BM, C), lambda i: (i, 0)),
                  pl.BlockSpec((BM, N), lambda i: (i, 0)),
                  pl.BlockSpec((4, C), lambda i: (0, 0))],
        out_specs=pl.BlockSpec((BM, C), lambda i: (i, 0)),
        out_shape=jax.ShapeDtypeStruct((M, C), jnp.float32),
    )(gath, xv, fc, r2, jnp.pad(sd, ((0, 1), (0, 0))))


def _comb_pool_body(gath_ref, o_ref):
    o_ref[...] = jnp.max(gath_ref[...], axis=1)


def _comb_pool(gath):
    M, N, C = gath.shape
    BM = _pick_bm(M, N, C)
    return pl.pallas_call(
        _comb_pool_body,
        grid=(M // BM,),
        in_specs=[pl.BlockSpec((BM, N, C), lambda i: (i, 0, 0))],
        out_specs=pl.BlockSpec((BM, C), lambda i: (i, 0)),
        out_shape=jax.ShapeDtypeStruct((M, C), jnp.float32),
    )(gath)


def _nearest_body(src_ref, tgt_ref, o_ref, *, S, V):
    b = pl.program_id(0)
    s = src_ref[0]
    tt = tgt_ref[0]
    inner = (s[:, 0:1] * tt[0:1, :] + s[:, 1:2] * tt[1:2, :]
             + s[:, 2:3] * tt[2:3, :])
    s2 = jnp.sum(s * s, axis=1, keepdims=True)
    t2 = jnp.sum(tt * tt, axis=0, keepdims=True)
    d = s2 + t2 - 2.0 * inner
    m = jnp.min(d, axis=0, keepdims=True)
    iS = lax.broadcasted_iota(jnp.int32, (S, V), 0)
    sel = jnp.where(d == m, iS, S)
    o_ref[0] = jnp.min(sel, axis=0, keepdims=True) + b * S


def _nearest(src, tgt):
    bs, S, _ = src.shape
    V = tgt.shape[1]
    tt = jnp.transpose(tgt, (0, 2, 1))
    return pl.pallas_call(
        functools.partial(_nearest_body, S=S, V=V),
        grid=(bs,),
        in_specs=[pl.BlockSpec((1, S, 3), lambda b: (b, 0, 0)),
                  pl.BlockSpec((1, 3, V), lambda b: (b, 0, 0))],
        out_specs=pl.BlockSpec((1, 1, V), lambda b: (b, 0, 0)),
        out_shape=jax.ShapeDtypeStruct((bs, 1, V), jnp.int32),
    )(src, tt)


def _rowmax_body(x_ref, o_ref):
    o_ref[0] = jnp.max(x_ref[0], axis=0, keepdims=True)


def _rowmax(x):
    bs, V, D = x.shape
    return pl.pallas_call(
        _rowmax_body,
        grid=(bs,),
        in_specs=[pl.BlockSpec((1, V, D), lambda b: (b, 0, 0))],
        out_specs=pl.BlockSpec((1, 1, D), lambda b: (b, 0, 0)),
        out_shape=jax.ShapeDtypeStruct((bs, 1, D), jnp.float32),
    )(x)[:, 0, :]


def _lsm_body(x_ref, o_ref):
    x = x_ref[...]
    m = jnp.max(x, axis=1, keepdims=True)
    sh = x - m
    o_ref[...] = sh - jnp.log(jnp.sum(jnp.exp(sh), axis=1, keepdims=True))


def _lsm(x):
    return pl.pallas_call(
        _lsm_body,
        out_shape=jax.ShapeDtypeStruct(x.shape, jnp.float32),
    )(x)


def _pick_chunk(per_w, D):
    for ch in (128, 96, 80, 64, 48, 40, 32, 24, 16, 8):
        if per_w % ch == 0 and ch * D <= 96 * 1024:
            return ch
    return 8


def _sc_gather(table, idx):
    T, D = table.shape
    (M,) = idx.shape
    per_w = M // _NW
    CH = _pick_chunk(per_w, D)
    nch = per_w // CH
    assert D % 128 == 0, "indirect-stream rows must be tile-aligned"
    mesh = plsc.VectorSubcoreMesh(core_axis_name="c", subcore_axis_name="s",
                                  num_cores=_NC, num_subcores=_NS)

    @functools.partial(
        pl.kernel,
        out_type=jax.ShapeDtypeStruct((M, D), jnp.float32),
        mesh=mesh,
        scratch_types=[pltpu.VMEM((CH,), jnp.int32),
                       pltpu.VMEM((CH, D), jnp.float32),
                       pltpu.SemaphoreType.DMA],
    )
    def gk(table_hbm, idx_hbm, out_hbm, idx_v, rows_v, sem):
        wid = lax.axis_index("s") * _NC + lax.axis_index("c")
        base = wid * per_w

        def body(i, carry):
            off = base + i * CH
            pltpu.sync_copy(idx_hbm.at[pl.ds(off, CH)], idx_v)
            pltpu.async_copy(table_hbm.at[idx_v], rows_v, sem).wait()
            pltpu.sync_copy(rows_v, out_hbm.at[pl.ds(off, CH)])
            return carry

        lax.fori_loop(0, nch, body, 0)

    return gk(table, idx)


def _normdir(sd):
    return sd / jnp.maximum(jnp.linalg.norm(sd, axis=0, keepdims=True), 1e-12)


def _conv_surface(xyz, xv4, r2, sd):
    return _comb_surface(xyz, xv4, r2, _normdir(sd))


def _conv_layer_pair(fmA, fmB, xyzA, xyzB, xv4, niA, niB, r2A, r2B,
                     wA, bA, sdA, wB, bB, sdB, vfp=None):
    C = sdA.shape[1]
    M = fmA.shape[0]
    foA = _mm(fmA, wA, bA)
    foB = _mm(fmB, wB, bB)
    nA = niA.shape[0]
    ni = jnp.concatenate([niA, niB + M])
    if vfp is None:
        tab = jnp.concatenate([foA[:, C:], foB[:, C:]], axis=0)
        g = _sc_gather(tab, ni)
        gathA = g[:nA].reshape(M, nA // M, C)
        gathB = g[nA:].reshape(M, -1, C)
        outA = _comb_layer(gathA, xyzA, xv4, foA[:, :C], r2A, _normdir(sdA))
        outB = _comb_layer(gathB, xyzB, xv4, foB[:, :C], r2B, _normdir(sdB))
        return outA, outB, xyzA, xyzB
    tab = jnp.concatenate(
        [jnp.concatenate([vfp, foA[:, C:]], axis=1),
         jnp.concatenate([vfp, foB[:, C:]], axis=1)], axis=0)
    g = _sc_gather(tab, ni)
    gathA = g[:nA].reshape(M, nA // M, 128 + C)
    gathB = g[nA:].reshape(M, -1, 128 + C)
    outA = _comb_layer_packed(gathA, xv4, foA[:, :C], r2A, _normdir(sdA), 128)
    outB = _comb_layer_packed(gathB, xv4, foB[:, :C], r2B, _normdir(sdB), 128)
    return outA, outB, gathA[:, :, :4], gathB[:, :, :4]


def kernel(vertices, onehot, params):
    p = params
    bs = vertices.shape[0]
    V = vertices.shape[2]
    verts = jnp.transpose(vertices, (0, 2, 1))
    M1 = bs * V
    vf = verts.reshape(M1, 3)

    idx51, d51 = _topk(verts, 51, 256)
    ni50 = idx51[:, :, 1:51].reshape(-1)
    r2_50 = d51[:, :, 1:51].reshape(M1, 50)
    ni10 = idx51[:, :, 1:11].reshape(-1)
    r2_10 = d51[:, :, 1:11].reshape(M1, 10)
    ni4 = idx51[:, :, 1:5].reshape(-1)

    vf4 = jnp.pad(vf, ((0, 0), (0, 1)))
    vfp1 = jnp.pad(vf, ((0, 0), (0, 125)))
    xyzb = _sc_gather(vfp1, ni50)
    xyz50 = xyzb.reshape(M1, 50, 128)[:, :, :4]
    xyz10 = xyz50[:, :10]

    fm0 = _bn(_conv_surface(xyz50, vf4, r2_50, p["conv0_dir"]),
              p["bn0_g"], p["bn0_b"])
    fm0l = _bn(_conv_surface(xyz10, vf4, r2_10, p["conv0l_dir"]),
               p["bn0l_g"], p["bn0l_b"])
    c1, c1l, _, _ = _conv_layer_pair(
        fm0, fm0l, xyz50, xyz10, vf4, ni50, ni10, r2_50, r2_10,
        p["conv1_w"], p["conv1_b"], p["conv1_dir"],
        p["conv1l_w"], p["conv1l_b"], p["conv1l_dir"])
    fm1 = _bn(c1, p["bn1_g"], p["bn1_b"])
    fm1l = _bn(c1l, p["bn1l_g"], p["bn1l_b"])
    fm1t = _mm(jnp.concatenate([fm1, fm1l], axis=1),
               p["down0_w"], p["down0_b"])

    V1 = V // 4
    perm1 = jax.random.permutation(jax.random.key(1), V)[:V1]
    pooled = _comb_pool(_sc_gather(fm1t, ni4).reshape(M1, 4, 128))
    fp1 = pooled.reshape(bs, V, 128)[:, perm1].reshape(bs * V1, 128)
    vp1 = verts[:, perm1]
    v1f = vp1.reshape(bs * V1, 3)
    M2 = bs * V1

    idx51_2, d51_2 = _topk(vp1, 51, V1)
    n2_50 = idx51_2[:, :, 1:51].reshape(-1)
    q2_50 = d51_2[:, :, 1:51].reshape(M2, 50)
    n2_10 = idx51_2[:, :, 1:11].reshape(-1)
    q2_10 = d51_2[:, :, 1:11].reshape(M2, 10)
    n2_4 = idx51_2[:, :, 1:5].reshape(-1)

    v1f4 = jnp.pad(v1f, ((0, 0), (0, 1)))
    vfp2 = jnp.pad(v1f, ((0, 0), (0, 125)))
    c2, c2l, xyz2_50, xyz2_10 = _conv_layer_pair(
        fp1, fp1, None, None, v1f4, n2_50, n2_10, q2_50, q2_10,
        p["conv2_w"], p["conv2_b"], p["conv2_dir"],
        p["conv2l_w"], p["conv2l_b"], p["conv2l_dir"], vfp=vfp2)
    fm2 = _bn(c2, p["bn2_g"], p["bn2_b"])
    fm2l = _bn(c2l, p["bn2l_g"], p["bn2l_b"])
    c3, c3l, _, _ = _conv_layer_pair(
        fm2, fm2l, xyz2_50, xyz2_10, v1f4, n2_50, n2_10, q2_50, q2_10,
        p["conv3_w"], p["conv3_b"], p["conv3_dir"],
        p["conv3l_w"], p["conv3l_b"], p["conv3l_dir"])
    fm3 = _bn(c3, p["bn3_g"], p["bn3_b"])
    fm3l = _bn(c3l, p["bn3l_g"], p["bn3l_b"])
    fm3t = _mm(jnp.concatenate([fm3, fm3l], axis=1),
               p["down1_w"], p["down1_b"])

    V2 = V1 // 4
    perm2 = jax.random.permutation(jax.random.key(2), V1)[:V2]
    pooled2 = _comb_pool(_sc_gather(fm3t, n2_4).reshape(M2, 4, 256))
    fp2 = pooled2.reshape(bs, V1, 256)[:, perm2].reshape(bs * V2, 256)
    vp2 = vp1[:, perm2]
    v2f = vp2.reshape(bs * V2, 3)
    M3 = bs * V2

    idx51_3, d51_3 = _topk(vp2, 51, V2)
    n3_50 = idx51_3[:, :, 1:51].reshape(-1)
    q3_50 = d51_3[:, :, 1:51].reshape(M3, 50)
    n3_10 = idx51_3[:, :, 1:11].reshape(-1)
    q3_10 = d51_3[:, :, 1:11].reshape(M3, 10)

    v2f4 = jnp.pad(v2f, ((0, 0), (0, 1)))
    vfp3 = jnp.pad(v2f, ((0, 0), (0, 125)))
    fm4, fm4l, _, _ = _conv_layer_pair(
        fp2, fp2, None, None, v2f4, n3_50, n3_10, q3_50, q3_10,
        p["conv4_w"], p["conv4_b"], p["conv4_dir"],
        p["conv4l_w"], p["conv4l_b"], p["conv4l_dir"], vfp=vfp3)
    fm4t = _mm(jnp.concatenate([fm4, fm4l], axis=1),
               p["down2_w"], p["down2_b"])

    fglob = _rowmax(fm4t.reshape(bs, V2, 512))

    np1 = _nearest(vp1, verts).reshape(-1)
    np2 = _nearest(vp2, verts).reshape(-1)
    tabn = jnp.concatenate([jnp.pad(fm3t, ((0, 0), (0, 256))), fm4t], axis=0)
    gn = _sc_gather(tabn, jnp.concatenate([np1, np2 + M2]))
    fm3f = gn[:M1, :256]
    fm4f = gn[M1:]
    fg = jnp.broadcast_to(fglob[:, None, :], (bs, V, 512)).reshape(M1, 512)
    oh = jnp.broadcast_to(onehot[:, None, :],
                          (bs, V, onehot.shape[1])).reshape(M1, -1)
    fuse = jnp.concatenate([fm1t, fm3f, fm4f, fg, oh], axis=1)

    x = _mm(fuse, p["h1_w"].T, p["h1_b"], act="relu")
    x = _mm(x, p["h2_w"].T, p["h2_b"], act="relu")
    x = _mm(x, p["h3_w"].T, p["h3_b"])
    return _lsm(x).reshape(bs, V, 50)


# speedup vs baseline: 1.2239x; 1.0382x over previous
import functools

import jax
import jax.numpy as jnp
from jax import lax
from jax.experimental import pallas as pl
from jax.experimental.pallas import tpu as pltpu
from jax.experimental.pallas import tpu_sc as plsc

_NC, _NS = 2, 16
_NW = _NC * _NS


def _topk_body(vr_ref, vt_ref, idx_ref, d_ref, *, K1, V):
    b = pl.program_id(0)
    vr = vr_ref[0]
    vt = vt_ref[0]
    R = vr.shape[0]
    inner = (vr[:, 0:1] * vt[0:1, :] + vr[:, 1:2] * vt[1:2, :]
             + vr[:, 2:3] * vt[2:3, :])
    qc = jnp.sum(vt * vt, axis=0, keepdims=True)
    qr = jnp.sum(vr * vr, axis=1, keepdims=True)
    neg = 2.0 * inner - qc - qr
    bits = lax.bitcast_convert_type(neg, jnp.int32)
    srt = jnp.where(bits >= 0, bits, bits ^ jnp.int32(0x7FFFFFFF))
    iota = lax.broadcasted_iota(jnp.int32, (R, V), 1)
    keys = (srt & jnp.int32(~0x7FF)) | (jnp.int32(2047) - iota)
    iok = lax.broadcasted_iota(jnp.int32, (R, K1), 1)
    kmin = jnp.int32(-0x80000000)

    def step(t, carry):
        keys, ak = carry
        m = jnp.max(keys, axis=1, keepdims=True)
        ak = jnp.where(iok == t, m, ak)
        keys = jnp.where(keys == m, kmin, keys)
        return keys, ak

    ak0 = jnp.zeros((R, K1), jnp.int32)
    _, ak = lax.fori_loop(0, K1, step, (keys, ak0))
    idx_ref[0] = (jnp.int32(2047) - (ak & jnp.int32(0x7FF))) + b * V
    vb = ak & jnp.int32(~0x7FF)
    vb = jnp.where(vb >= 0, vb, vb ^ jnp.int32(0x7FFFFFFF))
    d_ref[0] = -lax.bitcast_convert_type(vb, jnp.float32)


def _topk(verts, K1, R):
    bs, V, _ = verts.shape
    vt = jnp.transpose(verts, (0, 2, 1))
    return pl.pallas_call(
        functools.partial(_topk_body, K1=K1, V=V),
        grid=(bs, V // R),
        in_specs=[pl.BlockSpec((1, R, 3), lambda b, i: (b, i, 0)),
                  pl.BlockSpec((1, 3, V), lambda b, i: (b, 0, 0))],
        out_specs=[pl.BlockSpec((1, R, K1), lambda b, i: (b, i, 0)),
                   pl.BlockSpec((1, R, K1), lambda b, i: (b, i, 0))],
        out_shape=[jax.ShapeDtypeStruct((bs, V, K1), jnp.int32),
                   jax.ShapeDtypeStruct((bs, V, K1), jnp.float32)],
    )(verts, vt)


def _mm_body(x_ref, w_ref, b_ref, o_ref, *, act, K):
    x = x_ref[...]
    w = w_ref[...]
    if K <= 4:
        acc = x[:, 0:1] * w[0:1, :]
        for k in range(1, K):
            acc = acc + x[:, k:k + 1] * w[k:k + 1, :]
    else:
        acc = jnp.dot(x, w, preferred_element_type=jnp.float32)
    acc = acc + b_ref[...]
    if act == "relu":
        acc = jnp.maximum(acc, 0.0)
    o_ref[...] = acc


def _mm(x, w, b=None, act=None):
    M, K = x.shape
    N = w.shape[1]
    if b is None:
        b = jnp.zeros((N,), jnp.float32)
    BM = min(1024, M)
    return pl.pallas_call(
        functools.partial(_mm_body, act=act, K=K),
        grid=(M // BM,),
        in_specs=[pl.BlockSpec((BM, K), lambda i: (i, 0)),
                  pl.BlockSpec((K, N), lambda i: (0, 0)),
                  pl.BlockSpec((1, N), lambda i: (0, 0))],
        out_specs=pl.BlockSpec((BM, N), lambda i: (i, 0)),
        out_shape=jax.ShapeDtypeStruct((M, N), jnp.float32),
    )(x, w, b[None, :])


def _bn_body(x_ref, g_ref, b_ref, o_ref, *, relu):
    x = x_ref[...]
    m = jnp.mean(x, axis=0, keepdims=True)
    v = jnp.mean((x - m) ** 2, axis=0, keepdims=True)
    y = g_ref[...] * (x - m) / jnp.sqrt(v + 1e-5) + b_ref[...]
    if relu:
        y = jnp.maximum(y, 0.0)
    o_ref[...] = y


def _bn(x, g, b, relu=True):
    M, C = x.shape
    return pl.pallas_call(
        functools.partial(_bn_body, relu=relu),
        out_shape=jax.ShapeDtypeStruct((M, C), jnp.float32),
    )(x, g[None, :], b[None, :])


def _pick_bm(M, N, D):
    for bm in (1024, 512, 256, 128, 64, 32):
        if M % bm == 0 and bm * N * D * 4 <= 6 * 2**20:
            return bm
    return 32


def _rinv(r2):
    return 1.0 / jnp.maximum(jnp.sqrt(jnp.maximum(r2, 0.0)), 1e-12)


def _theta(xyz, xv, r2, sd4):
    BM, N = r2.shape
    C = sd4.shape[1]
    ri = _rinv(r2)[:, :, None]
    diff = xyz[:, :, :4] - xv[:, None, :]
    scaled = diff * ri
    dot = jnp.dot(scaled.reshape(BM * N, 4), sd4,
                  preferred_element_type=jnp.float32).reshape(BM, N, C)
    return jnp.maximum(dot, 0.0)


def _comb_surface_body(xyz_ref, xv_ref, r2_ref, sd_ref, o_ref):
    th = _theta(xyz_ref[...], xv_ref[...], r2_ref[...], sd_ref[...])
    o_ref[...] = jnp.max(th, axis=1)


def _comb_surface(xyz, xv, r2, sd):
    M, N, W = xyz.shape
    C = sd.shape[1]
    BM = _pick_bm(M, N, 128 + C)
    return pl.pallas_call(
        _comb_surface_body,
        grid=(M // BM,),
        in_specs=[pl.BlockSpec((BM, N, W), lambda i: (i, 0, 0)),
                  pl.BlockSpec((BM, 4), lambda i: (i, 0)),
                  pl.BlockSpec((BM, N), lambda i: (i, 0)),
                  pl.BlockSpec((4, C), lambda i: (0, 0))],
        out_specs=pl.BlockSpec((BM, C), lambda i: (i, 0)),
        out_shape=jax.ShapeDtypeStruct((M, C), jnp.float32),
    )(xyz, xv, r2, jnp.pad(sd, ((0, 1), (0, 0))))


def _comb_layer_body(fs_ref, xyz_ref, xv_ref, fc_ref, r2_ref, sd_ref, o_ref):
    th = _theta(xyz_ref[...], xv_ref[...], r2_ref[...], sd_ref[...])
    o_ref[...] = fc_ref[...] + jnp.max(th * fs_ref[...], axis=1)


def _comb_layer(fs, xyz, xv, fc, r2, sd):
    M, N, C = fs.shape
    W = xyz.shape[2]
    BM = _pick_bm(M, N, C + 256)
    return pl.pallas_call(
        _comb_layer_body,
        grid=(M // BM,),
        in_specs=[pl.BlockSpec((BM, N, C), lambda i: (i, 0, 0)),
                  pl.BlockSpec((BM, N, W), lambda i: (i, 0, 0)),
                  pl.BlockSpec((BM, 4), lambda i: (i, 0)),
                  pl.BlockSpec((BM, C), lambda i: (i, 0)),
                  pl.BlockSpec((BM, N), lambda i: (i, 0)),
                  pl.BlockSpec((4, C), lambda i: (0, 0))],
        out_specs=pl.BlockSpec((BM, C), lambda i: (i, 0)),
        out_shape=jax.ShapeDtypeStruct((M, C), jnp.float32),
    )(fs, xyz, xv, fc, r2, jnp.pad(sd, ((0, 1), (0, 0))))


def _comb_layer_packed_body(g_ref, xv_ref, fc_ref, r2_ref, sd_ref, o_ref, *,
                            fsoff):
    x = g_ref[...]
    th = _theta(x, xv_ref[...], r2_ref[...], sd_ref[...])
    o_ref[...] = fc_ref[...] + jnp.max(th * x[:, :, fsoff:], axis=1)


def _comb_layer_packed(gath, xv, fc, r2, sd, fsoff):
    M, N, D = gath.shape
    C = D - fsoff
    BM = _pick_bm(M, N, D + C)
    return pl.pallas_call(
        functools.partial(_comb_layer_packed_body, fsoff=fsoff),
        grid=(M // BM,),
        in_specs=[pl.BlockSpec((BM, N, D), lambda i: (i, 0, 0)),
                  pl.BlockSpec((BM, 4), lambda i: (i, 0)),
                  pl.BlockSpec((BM, C), lambda i: (i, 0)),
                  pl.BlockSpec((BM, N), lambda i: (i, 0)),
                  pl.BlockSpec((4, C), lambda i: (0, 0))],
        out_specs=pl.BlockSpec((BM, C), lambda i: (i, 0)),
        out_shape=jax.ShapeDtypeStruct((M, C), jnp.float32),
    )(gath, xv, fc, r2, jnp.pad(sd, ((0, 1), (0, 0))))


def _comb_pool_body(gath_ref, o_ref):
    o_ref[...] = jnp.max(gath_ref[...], axis=1)


def _comb_pool(gath):
    M, N, C = gath.shape
    BM = _pick_bm(M, N, C)
    return pl.pallas_call(
        _comb_pool_body,
        grid=(M // BM,),
        in_specs=[pl.BlockSpec((BM, N, C), lambda i: (i, 0, 0))],
        out_specs=pl.BlockSpec((BM, C), lambda i: (i, 0)),
        out_shape=jax.ShapeDtypeStruct((M, C), jnp.float32),
    )(gath)


def _nearest_body(src_ref, tgt_ref, o_ref, *, S, V):
    b = pl.program_id(0)
    s = src_ref[0]
    tt = tgt_ref[0]
    inner = (s[:, 0:1] * tt[0:1, :] + s[:, 1:2] * tt[1:2, :]
             + s[:, 2:3] * tt[2:3, :])
    s2 = jnp.sum(s * s, axis=1, keepdims=True)
    t2 = jnp.sum(tt * tt, axis=0, keepdims=True)
    d = s2 + t2 - 2.0 * inner
    m = jnp.min(d, axis=0, keepdims=True)
    iS = lax.broadcasted_iota(jnp.int32, (S, V), 0)
    sel = jnp.where(d == m, iS, S)
    o_ref[0] = jnp.min(sel, axis=0, keepdims=True) + b * S


def _nearest(src, tgt):
    bs, S, _ = src.shape
    V = tgt.shape[1]
    tt = jnp.transpose(tgt, (0, 2, 1))
    return pl.pallas_call(
        functools.partial(_nearest_body, S=S, V=V),
        grid=(bs,),
        in_specs=[pl.BlockSpec((1, S, 3), lambda b: (b, 0, 0)),
                  pl.BlockSpec((1, 3, V), lambda b: (b, 0, 0))],
        out_specs=pl.BlockSpec((1, 1, V), lambda b: (b, 0, 0)),
        out_shape=jax.ShapeDtypeStruct((bs, 1, V), jnp.int32),
    )(src, tt)


def _rowmax_body(x_ref, o_ref):
    o_ref[0] = jnp.max(x_ref[0], axis=0, keepdims=True)


def _rowmax(x):
    bs, V, D = x.shape
    return pl.pallas_call(
        _rowmax_body,
        grid=(bs,),
        in_specs=[pl.BlockSpec((1, V, D), lambda b: (b, 0, 0))],
        out_specs=pl.BlockSpec((1, 1, D), lambda b: (b, 0, 0)),
        out_shape=jax.ShapeDtypeStruct((bs, 1, D), jnp.float32),
    )(x)[:, 0, :]


def _lsm_body(x_ref, o_ref):
    x = x_ref[...]
    m = jnp.max(x, axis=1, keepdims=True)
    sh = x - m
    o_ref[...] = sh - jnp.log(jnp.sum(jnp.exp(sh), axis=1, keepdims=True))


def _lsm(x):
    return pl.pallas_call(
        _lsm_body,
        out_shape=jax.ShapeDtypeStruct(x.shape, jnp.float32),
    )(x)


def _pick_chunk(per_w, D):
    for ch in (128, 96, 80, 64, 48, 40, 32, 24, 16, 8):
        if per_w % ch == 0 and ch * D <= 48 * 1024:
            return ch
    return 8


def _sc_gather(table, idx):
    T, D = table.shape
    (M,) = idx.shape
    per_w = M // _NW
    CH = _pick_chunk(per_w, D)
    nch = per_w // CH
    assert D % 128 == 0, "indirect-stream rows must be tile-aligned"
    mesh = plsc.VectorSubcoreMesh(core_axis_name="c", subcore_axis_name="s",
                                  num_cores=_NC, num_subcores=_NS)

    @functools.partial(
        pl.kernel,
        out_type=jax.ShapeDtypeStruct((M, D), jnp.float32),
        mesh=mesh,
        scratch_types=[pltpu.VMEM((CH,), jnp.int32),
                       pltpu.VMEM((CH,), jnp.int32),
                       pltpu.VMEM((CH, D), jnp.float32),
                       pltpu.VMEM((CH, D), jnp.float32),
                       pltpu.SemaphoreType.DMA,
                       pltpu.SemaphoreType.DMA],
    )
    def gk(table_hbm, idx_hbm, out_hbm, idx0, idx1, rows0, rows1, s0, s1):
        wid = lax.axis_index("s") * _NC + lax.axis_index("c")
        base = wid * per_w

        def one(off, idx_v, rows_v, sem):
            pltpu.sync_copy(idx_hbm.at[pl.ds(off, CH)], idx_v)
            return pltpu.async_copy(table_hbm.at[idx_v], rows_v, sem)

        def body(i, carry):
            o0 = base + 2 * i * CH
            c0 = one(o0, idx0, rows0, s0)
            c1 = one(o0 + CH, idx1, rows1, s1)
            c0.wait()
            pltpu.sync_copy(rows0, out_hbm.at[pl.ds(o0, CH)])
            c1.wait()
            pltpu.sync_copy(rows1, out_hbm.at[pl.ds(o0 + CH, CH)])
            return carry

        lax.fori_loop(0, nch // 2, body, 0)
        if nch % 2:
            off = base + (nch - 1) * CH
            one(off, idx0, rows0, s0).wait()
            pltpu.sync_copy(rows0, out_hbm.at[pl.ds(off, CH)])

    return gk(table, idx)


def _normdir(sd):
    return sd / jnp.maximum(jnp.linalg.norm(sd, axis=0, keepdims=True), 1e-12)


def _conv_surface(xyz, xv4, r2, sd):
    return _comb_surface(xyz, xv4, r2, _normdir(sd))


def _conv_layer_pair(fmA, fmB, xyzA, xyzB, xv4, niA, niB, r2A, r2B,
                     wA, bA, sdA, wB, bB, sdB, vfp=None):
    C = sdA.shape[1]
    M = fmA.shape[0]
    foA = _mm(fmA, wA, bA)
    foB = _mm(fmB, wB, bB)
    nA = niA.shape[0]
    ni = jnp.concatenate([niA, niB + M])
    if vfp is None:
        tab = jnp.concatenate([foA[:, C:], foB[:, C:]], axis=0)
        g = _sc_gather(tab, ni)
        gathA = g[:nA].reshape(M, nA // M, C)
        gathB = g[nA:].reshape(M, -1, C)
        outA = _comb_layer(gathA, xyzA, xv4, foA[:, :C], r2A, _normdir(sdA))
        outB = _comb_layer(gathB, xyzB, xv4, foB[:, :C], r2B, _normdir(sdB))
        return outA, outB, xyzA, xyzB
    tab = jnp.concatenate(
        [jnp.concatenate([vfp, foA[:, C:]], axis=1),
         jnp.concatenate([vfp, foB[:, C:]], axis=1)], axis=0)
    g = _sc_gather(tab, ni)
    gathA = g[:nA].reshape(M, nA // M, 128 + C)
    gathB = g[nA:].reshape(M, -1, 128 + C)
    outA = _comb_layer_packed(gathA, xv4, foA[:, :C], r2A, _normdir(sdA), 128)
    outB = _comb_layer_packed(gathB, xv4, foB[:, :C], r2B, _normdir(sdB), 128)
    return outA, outB, gathA[:, :, :4], gathB[:, :, :4]


def kernel(vertices, onehot, params):
    p = params
    bs = vertices.shape[0]
    V = vertices.shape[2]
    verts = jnp.transpose(vertices, (0, 2, 1))
    M1 = bs * V
    vf = verts.reshape(M1, 3)

    idx51, d51 = _topk(verts, 51, 256)
    ni50 = idx51[:, :, 1:51].reshape(-1)
    r2_50 = d51[:, :, 1:51].reshape(M1, 50)
    ni10 = idx51[:, :, 1:11].reshape(-1)
    r2_10 = d51[:, :, 1:11].reshape(M1, 10)
    ni4 = idx51[:, :, 1:5].reshape(-1)

    vf4 = jnp.pad(vf, ((0, 0), (0, 1)))
    vfp1 = jnp.pad(vf, ((0, 0), (0, 125)))
    xyzb = _sc_gather(vfp1, ni50)
    xyz50 = xyzb.reshape(M1, 50, 128)[:, :, :4]
    xyz10 = xyz50[:, :10]

    fm0 = _bn(_conv_surface(xyz50, vf4, r2_50, p["conv0_dir"]),
              p["bn0_g"], p["bn0_b"])
    fm0l = _bn(_conv_surface(xyz10, vf4, r2_10, p["conv0l_dir"]),
               p["bn0l_g"], p["bn0l_b"])
    c1, c1l, _, _ = _conv_layer_pair(
        fm0, fm0l, xyz50, xyz10, vf4, ni50, ni10, r2_50, r2_10,
        p["conv1_w"], p["conv1_b"], p["conv1_dir"],
        p["conv1l_w"], p["conv1l_b"], p["conv1l_dir"])
    fm1 = _bn(c1, p["bn1_g"], p["bn1_b"])
    fm1l = _bn(c1l, p["bn1l_g"], p["bn1l_b"])
    fm1t = _mm(jnp.concatenate([fm1, fm1l], axis=1),
               p["down0_w"], p["down0_b"])

    V1 = V // 4
    perm1 = jax.random.permutation(jax.random.key(1), V)[:V1]
    pooled = _comb_pool(_sc_gather(fm1t, ni4).reshape(M1, 4, 128))
    fp1 = pooled.reshape(bs, V, 128)[:, perm1].reshape(bs * V1, 128)
    vp1 = verts[:, perm1]
    v1f = vp1.reshape(bs * V1, 3)
    M2 = bs * V1

    idx51_2, d51_2 = _topk(vp1, 51, V1)
    n2_50 = idx51_2[:, :, 1:51].reshape(-1)
    q2_50 = d51_2[:, :, 1:51].reshape(M2, 50)
    n2_10 = idx51_2[:, :, 1:11].reshape(-1)
    q2_10 = d51_2[:, :, 1:11].reshape(M2, 10)
    n2_4 = idx51_2[:, :, 1:5].reshape(-1)

    v1f4 = jnp.pad(v1f, ((0, 0), (0, 1)))
    vfp2 = jnp.pad(v1f, ((0, 0), (0, 125)))
    c2, c2l, xyz2_50, xyz2_10 = _conv_layer_pair(
        fp1, fp1, None, None, v1f4, n2_50, n2_10, q2_50, q2_10,
        p["conv2_w"], p["conv2_b"], p["conv2_dir"],
        p["conv2l_w"], p["conv2l_b"], p["conv2l_dir"], vfp=vfp2)
    fm2 = _bn(c2, p["bn2_g"], p["bn2_b"])
    fm2l = _bn(c2l, p["bn2l_g"], p["bn2l_b"])
    c3, c3l, _, _ = _conv_layer_pair(
        fm2, fm2l, xyz2_50, xyz2_10, v1f4, n2_50, n2_10, q2_50, q2_10,
        p["conv3_w"], p["conv3_b"], p["conv3_dir"],
        p["conv3l_w"], p["conv3l_b"], p["conv3l_dir"])
    fm3 = _bn(c3, p["bn3_g"], p["bn3_b"])
    fm3l = _bn(c3l, p["bn3l_g"], p["bn3l_b"])
    fm3t = _mm(jnp.concatenate([fm3, fm3l], axis=1),
               p["down1_w"], p["down1_b"])

    V2 = V1 // 4
    perm2 = jax.random.permutation(jax.random.key(2), V1)[:V2]
    pooled2 = _comb_pool(_sc_gather(fm3t, n2_4).reshape(M2, 4, 256))
    fp2 = pooled2.reshape(bs, V1, 256)[:, perm2].reshape(bs * V2, 256)
    vp2 = vp1[:, perm2]
    v2f = vp2.reshape(bs * V2, 3)
    M3 = bs * V2

    idx51_3, d51_3 = _topk(vp2, 51, V2)
    n3_50 = idx51_3[:, :, 1:51].reshape(-1)
    q3_50 = d51_3[:, :, 1:51].reshape(M3, 50)
    n3_10 = idx51_3[:, :, 1:11].reshape(-1)
    q3_10 = d51_3[:, :, 1:11].reshape(M3, 10)

    v2f4 = jnp.pad(v2f, ((0, 0), (0, 1)))
    vfp3 = jnp.pad(v2f, ((0, 0), (0, 125)))
    fm4, fm4l, _, _ = _conv_layer_pair(
        fp2, fp2, None, None, v2f4, n3_50, n3_10, q3_50, q3_10,
        p["conv4_w"], p["conv4_b"], p["conv4_dir"],
        p["conv4l_w"], p["conv4l_b"], p["conv4l_dir"], vfp=vfp3)
    fm4t = _mm(jnp.concatenate([fm4, fm4l], axis=1),
               p["down2_w"], p["down2_b"])

    fglob = _rowmax(fm4t.reshape(bs, V2, 512))

    np1 = _nearest(vp1, verts).reshape(-1)
    np2 = _nearest(vp2, verts).reshape(-1)
    tabn = jnp.concatenate([jnp.pad(fm3t, ((0, 0), (0, 256))), fm4t], axis=0)
    gn = _sc_gather(tabn, jnp.concatenate([np1, np2 + M2]))
    fm3f = gn[:M1, :256]
    fm4f = gn[M1:]
    fg = jnp.broadcast_to(fglob[:, None, :], (bs, V, 512)).reshape(M1, 512)
    oh = jnp.broadcast_to(onehot[:, None, :],
                          (bs, V, onehot.shape[1])).reshape(M1, -1)
    fuse = jnp.concatenate([fm1t, fm3f, fm4f, fg, oh], axis=1)

    x = _mm(fuse, p["h1_w"].T, p["h1_b"], act="relu")
    x = _mm(x, p["h2_w"].T, p["h2_b"], act="relu")
    x = _mm(x, p["h3_w"].T, p["h3_b"])
    return _lsm(x).reshape(bs, V, 50)


# speedup vs baseline: 1.2320x; 1.0066x over previous
import functools

import jax
import jax.numpy as jnp
from jax import lax
from jax.experimental import pallas as pl
from jax.experimental.pallas import tpu as pltpu
from jax.experimental.pallas import tpu_sc as plsc

_NC, _NS = 2, 16
_NW = _NC * _NS


def _topk_body(vr_ref, vt_ref, idx_ref, d_ref, *, K1, V):
    b = pl.program_id(0)
    vr = vr_ref[0]
    vt = vt_ref[0]
    R = vr.shape[0]
    inner = (vr[:, 0:1] * vt[0:1, :] + vr[:, 1:2] * vt[1:2, :]
             + vr[:, 2:3] * vt[2:3, :])
    qc = jnp.sum(vt * vt, axis=0, keepdims=True)
    qr = jnp.sum(vr * vr, axis=1, keepdims=True)
    neg = 2.0 * inner - qc - qr
    bits = lax.bitcast_convert_type(neg, jnp.int32)
    srt = jnp.where(bits >= 0, bits, bits ^ jnp.int32(0x7FFFFFFF))
    iota = lax.broadcasted_iota(jnp.int32, (R, V), 1)
    keys = (srt & jnp.int32(~0x7FF)) | (jnp.int32(2047) - iota)
    iok = lax.broadcasted_iota(jnp.int32, (R, K1), 1)
    kmin = jnp.int32(-0x80000000)

    def step(t, carry):
        keys, ak = carry
        m = jnp.max(keys, axis=1, keepdims=True)
        ak = jnp.where(iok == t, m, ak)
        keys = jnp.where(keys == m, kmin, keys)
        return keys, ak

    ak0 = jnp.zeros((R, K1), jnp.int32)
    _, ak = lax.fori_loop(0, K1, step, (keys, ak0))
    idx_ref[0] = (jnp.int32(2047) - (ak & jnp.int32(0x7FF))) + b * V
    vb = ak & jnp.int32(~0x7FF)
    vb = jnp.where(vb >= 0, vb, vb ^ jnp.int32(0x7FFFFFFF))
    d_ref[0] = -lax.bitcast_convert_type(vb, jnp.float32)


def _topk(verts, K1, R):
    bs, V, _ = verts.shape
    vt = jnp.transpose(verts, (0, 2, 1))
    return pl.pallas_call(
        functools.partial(_topk_body, K1=K1, V=V),
        grid=(bs, V // R),
        in_specs=[pl.BlockSpec((1, R, 3), lambda b, i: (b, i, 0)),
                  pl.BlockSpec((1, 3, V), lambda b, i: (b, 0, 0))],
        out_specs=[pl.BlockSpec((1, R, K1), lambda b, i: (b, i, 0)),
                   pl.BlockSpec((1, R, K1), lambda b, i: (b, i, 0))],
        out_shape=[jax.ShapeDtypeStruct((bs, V, K1), jnp.int32),
                   jax.ShapeDtypeStruct((bs, V, K1), jnp.float32)],
    )(verts, vt)


def _mm_body(x_ref, w_ref, b_ref, o_ref, *, act, K):
    x = x_ref[...]
    w = w_ref[...]
    if K <= 4:
        acc = x[:, 0:1] * w[0:1, :]
        for k in range(1, K):
            acc = acc + x[:, k:k + 1] * w[k:k + 1, :]
    else:
        acc = jnp.dot(x, w, preferred_element_type=jnp.float32)
    acc = acc + b_ref[...]
    if act == "relu":
        acc = jnp.maximum(acc, 0.0)
    o_ref[...] = acc


def _mm(x, w, b=None, act=None):
    M, K = x.shape
    N = w.shape[1]
    if b is None:
        b = jnp.zeros((N,), jnp.float32)
    BM = min(1024, M)
    return pl.pallas_call(
        functools.partial(_mm_body, act=act, K=K),
        grid=(M // BM,),
        in_specs=[pl.BlockSpec((BM, K), lambda i: (i, 0)),
                  pl.BlockSpec((K, N), lambda i: (0, 0)),
                  pl.BlockSpec((1, N), lambda i: (0, 0))],
        out_specs=pl.BlockSpec((BM, N), lambda i: (i, 0)),
        out_shape=jax.ShapeDtypeStruct((M, N), jnp.float32),
    )(x, w, b[None, :])


def _bn_body(x_ref, g_ref, b_ref, o_ref, *, relu):
    x = x_ref[...]
    m = jnp.mean(x, axis=0, keepdims=True)
    v = jnp.mean((x - m) ** 2, axis=0, keepdims=True)
    y = g_ref[...] * (x - m) / jnp.sqrt(v + 1e-5) + b_ref[...]
    if relu:
        y = jnp.maximum(y, 0.0)
    o_ref[...] = y


def _bn(x, g, b, relu=True):
    M, C = x.shape
    return pl.pallas_call(
        functools.partial(_bn_body, relu=relu),
        out_shape=jax.ShapeDtypeStruct((M, C), jnp.float32),
    )(x, g[None, :], b[None, :])


def _pick_bm(M, N, D):
    for bm in (1024, 512, 256, 128, 64, 32):
        if M % bm == 0 and bm * N * D * 4 <= 12 * 2**20:
            return bm
    return 32


def _rinv(r2):
    return 1.0 / jnp.maximum(jnp.sqrt(jnp.maximum(r2, 0.0)), 1e-12)


def _theta(xyz, xv, r2, sd4):
    BM, N = r2.shape
    C = sd4.shape[1]
    ri = _rinv(r2)[:, :, None]
    diff = xyz[:, :, :4] - xv[:, None, :]
    scaled = diff * ri
    dot = jnp.dot(scaled.reshape(BM * N, 4), sd4,
                  preferred_element_type=jnp.float32).reshape(BM, N, C)
    return jnp.maximum(dot, 0.0)


def _comb_surface_body(xyz_ref, xv_ref, r2_ref, sd_ref, o_ref):
    th = _theta(xyz_ref[...], xv_ref[...], r2_ref[...], sd_ref[...])
    o_ref[...] = jnp.max(th, axis=1)


def _comb_surface(xyz, xv, r2, sd):
    M, N, W = xyz.shape
    C = sd.shape[1]
    BM = _pick_bm(M, N, 128 + C)
    return pl.pallas_call(
        _comb_surface_body,
        grid=(M // BM,),
        in_specs=[pl.BlockSpec((BM, N, W), lambda i: (i, 0, 0)),
                  pl.BlockSpec((BM, 4), lambda i: (i, 0)),
                  pl.BlockSpec((BM, N), lambda i: (i, 0)),
                  pl.BlockSpec((4, C), lambda i: (0, 0))],
        out_specs=pl.BlockSpec((BM, C), lambda i: (i, 0)),
        out_shape=jax.ShapeDtypeStruct((M, C), jnp.float32),
    )(xyz, xv, r2, jnp.pad(sd, ((0, 1), (0, 0))))


def _comb_layer_body(fs_ref, xyz_ref, xv_ref, fc_ref, r2_ref, sd_ref, o_ref):
    th = _theta(xyz_ref[...], xv_ref[...], r2_ref[...], sd_ref[...])
    o_ref[...] = fc_ref[...] + jnp.max(th * fs_ref[...], axis=1)


def _comb_layer(fs, xyz, xv, fc, r2, sd):
    M, N, C = fs.shape
    W = xyz.shape[2]
    BM = _pick_bm(M, N, C + 256)
    return pl.pallas_call(
        _comb_layer_body,
        grid=(M // BM,),
        in_specs=[pl.BlockSpec((BM, N, C), lambda i: (i, 0, 0)),
                  pl.BlockSpec((BM, N, W), lambda i: (i, 0, 0)),
                  pl.BlockSpec((BM, 4), lambda i: (i, 0)),
                  pl.BlockSpec((BM, C), lambda i: (i, 0)),
                  pl.BlockSpec((BM, N), lambda i: (i, 0)),
                  pl.BlockSpec((4, C), lambda i: (0, 0))],
        out_specs=pl.BlockSpec((BM, C), lambda i: (i, 0)),
        out_shape=jax.ShapeDtypeStruct((M, C), jnp.float32),
    )(fs, xyz, xv, fc, r2, jnp.pad(sd, ((0, 1), (0, 0))))


def _comb_layer_packed_body(g_ref, xv_ref, fc_ref, r2_ref, sd_ref, o_ref, *,
                            fsoff):
    x = g_ref[...]
    th = _theta(x, xv_ref[...], r2_ref[...], sd_ref[...])
    o_ref[...] = fc_ref[...] + jnp.max(th * x[:, :, fsoff:], axis=1)


def _comb_layer_packed(gath, xv, fc, r2, sd, fsoff):
    M, N, D = gath.shape
    C = D - fsoff
    BM = _pick_bm(M, N, D + C)
    return pl.pallas_call(
        functools.partial(_comb_layer_packed_body, fsoff=fsoff),
        grid=(M // BM,),
        in_specs=[pl.BlockSpec((BM, N, D), lambda i: (i, 0, 0)),
                  pl.BlockSpec((BM, 4), lambda i: (i, 0)),
                  pl.BlockSpec((BM, C), lambda i: (i, 0)),
                  pl.BlockSpec((BM, N), lambda i: (i, 0)),
                  pl.BlockSpec((4, C), lambda i: (0, 0))],
        out_specs=pl.BlockSpec((BM, C), lambda i: (i, 0)),
        out_shape=jax.ShapeDtypeStruct((M, C), jnp.float32),
    )(gath, xv, fc, r2, jnp.pad(sd, ((0, 1), (0, 0))))


def _comb_pool_body(gath_ref, o_ref):
    o_ref[...] = jnp.max(gath_ref[...], axis=1)


def _comb_pool(gath):
    M, N, C = gath.shape
    BM = _pick_bm(M, N, C)
    return pl.pallas_call(
        _comb_pool_body,
        grid=(M // BM,),
        in_specs=[pl.BlockSpec((BM, N, C), lambda i: (i, 0, 0))],
        out_specs=pl.BlockSpec((BM, C), lambda i: (i, 0)),
        out_shape=jax.ShapeDtypeStruct((M, C), jnp.float32),
    )(gath)


def _nearest_body(src_ref, tgt_ref, o_ref, *, S, V):
    b = pl.program_id(0)
    s = src_ref[0]
    tt = tgt_ref[0]
    inner = (s[:, 0:1] * tt[0:1, :] + s[:, 1:2] * tt[1:2, :]
             + s[:, 2:3] * tt[2:3, :])
    s2 = jnp.sum(s * s, axis=1, keepdims=True)
    t2 = jnp.sum(tt * tt, axis=0, keepdims=True)
    d = s2 + t2 - 2.0 * inner
    m = jnp.min(d, axis=0, keepdims=True)
    iS = lax.broadcasted_iota(jnp.int32, (S, V), 0)
    sel = jnp.where(d == m, iS, S)
    o_ref[0] = jnp.min(sel, axis=0, keepdims=True) + b * S


def _nearest(src, tgt):
    bs, S, _ = src.shape
    V = tgt.shape[1]
    tt = jnp.transpose(tgt, (0, 2, 1))
    return pl.pallas_call(
        functools.partial(_nearest_body, S=S, V=V),
        grid=(bs,),
        in_specs=[pl.BlockSpec((1, S, 3), lambda b: (b, 0, 0)),
                  pl.BlockSpec((1, 3, V), lambda b: (b, 0, 0))],
        out_specs=pl.BlockSpec((1, 1, V), lambda b: (b, 0, 0)),
        out_shape=jax.ShapeDtypeStruct((bs, 1, V), jnp.int32),
    )(src, tt)


def _rowmax_body(x_ref, o_ref):
    o_ref[0] = jnp.max(x_ref[0], axis=0, keepdims=True)


def _rowmax(x):
    bs, V, D = x.shape
    return pl.pallas_call(
        _rowmax_body,
        grid=(bs,),
        in_specs=[pl.BlockSpec((1, V, D), lambda b: (b, 0, 0))],
        out_specs=pl.BlockSpec((1, 1, D), lambda b: (b, 0, 0)),
        out_shape=jax.ShapeDtypeStruct((bs, 1, D), jnp.float32),
    )(x)[:, 0, :]


def _lsm_body(x_ref, o_ref):
    x = x_ref[...]
    m = jnp.max(x, axis=1, keepdims=True)
    sh = x - m
    o_ref[...] = sh - jnp.log(jnp.sum(jnp.exp(sh), axis=1, keepdims=True))


def _lsm(x):
    return pl.pallas_call(
        _lsm_body,
        out_shape=jax.ShapeDtypeStruct(x.shape, jnp.float32),
    )(x)


def _pick_chunk(per_w, D):
    for ch in (128, 96, 80, 64, 48, 40, 32, 24, 16, 8):
        if per_w % ch == 0 and ch * D <= 48 * 1024:
            return ch
    return 8


def _sc_gather(table, idx):
    T, D = table.shape
    (M,) = idx.shape
    per_w = M // _NW
    CH = _pick_chunk(per_w, D)
    nch = per_w // CH
    assert D % 128 == 0, "indirect-stream rows must be tile-aligned"
    mesh = plsc.VectorSubcoreMesh(core_axis_name="c", subcore_axis_name="s",
                                  num_cores=_NC, num_subcores=_NS)

    @functools.partial(
        pl.kernel,
        out_type=jax.ShapeDtypeStruct((M, D), jnp.float32),
        mesh=mesh,
        scratch_types=[pltpu.VMEM((CH,), jnp.int32),
                       pltpu.VMEM((CH,), jnp.int32),
                       pltpu.VMEM((CH, D), jnp.float32),
                       pltpu.VMEM((CH, D), jnp.float32),
                       pltpu.SemaphoreType.DMA,
                       pltpu.SemaphoreType.DMA],
    )
    def gk(table_hbm, idx_hbm, out_hbm, idx0, idx1, rows0, rows1, s0, s1):
        wid = lax.axis_index("s") * _NC + lax.axis_index("c")
        base = wid * per_w

        def one(off, idx_v, rows_v, sem):
            pltpu.sync_copy(idx_hbm.at[pl.ds(off, CH)], idx_v)
            return pltpu.async_copy(table_hbm.at[idx_v], rows_v, sem)

        def body(i, carry):
            o0 = base + 2 * i * CH
            c0 = one(o0, idx0, rows0, s0)
            c1 = one(o0 + CH, idx1, rows1, s1)
            c0.wait()
            pltpu.sync_copy(rows0, out_hbm.at[pl.ds(o0, CH)])
            c1.wait()
            pltpu.sync_copy(rows1, out_hbm.at[pl.ds(o0 + CH, CH)])
            return carry

        lax.fori_loop(0, nch // 2, body, 0)
        if nch % 2:
            off = base + (nch - 1) * CH
            one(off, idx0, rows0, s0).wait()
            pltpu.sync_copy(rows0, out_hbm.at[pl.ds(off, CH)])

    return gk(table, idx)


def _normdir(sd):
    return sd / jnp.maximum(jnp.linalg.norm(sd, axis=0, keepdims=True), 1e-12)


def _conv_surface(xyz, xv4, r2, sd):
    return _comb_surface(xyz, xv4, r2, _normdir(sd))


def _conv_layer_pair(fmA, fmB, xyzA, xyzB, xv4, niA, niB, r2A, r2B,
                     wA, bA, sdA, wB, bB, sdB, vfp=None):
    C = sdA.shape[1]
    M = fmA.shape[0]
    foA = _mm(fmA, wA, bA)
    foB = _mm(fmB, wB, bB)
    nA = niA.shape[0]
    ni = jnp.concatenate([niA, niB + M])
    if vfp is None:
        tab = jnp.concatenate([foA[:, C:], foB[:, C:]], axis=0)
        g = _sc_gather(tab, ni)
        gathA = g[:nA].reshape(M, nA // M, C)
        gathB = g[nA:].reshape(M, -1, C)
        outA = _comb_layer(gathA, xyzA, xv4, foA[:, :C], r2A, _normdir(sdA))
        outB = _comb_layer(gathB, xyzB, xv4, foB[:, :C], r2B, _normdir(sdB))
        return outA, outB, xyzA, xyzB
    tab = jnp.concatenate(
        [jnp.concatenate([vfp, foA[:, C:]], axis=1),
         jnp.concatenate([vfp, foB[:, C:]], axis=1)], axis=0)
    g = _sc_gather(tab, ni)
    gathA = g[:nA].reshape(M, nA // M, 128 + C)
    gathB = g[nA:].reshape(M, -1, 128 + C)
    outA = _comb_layer_packed(gathA, xv4, foA[:, :C], r2A, _normdir(sdA), 128)
    outB = _comb_layer_packed(gathB, xv4, foB[:, :C], r2B, _normdir(sdB), 128)
    return outA, outB, gathA[:, :, :4], gathB[:, :, :4]


def kernel(vertices, onehot, params):
    p = params
    bs = vertices.shape[0]
    V = vertices.shape[2]
    verts = jnp.transpose(vertices, (0, 2, 1))
    M1 = bs * V
    vf = verts.reshape(M1, 3)

    idx51, d51 = _topk(verts, 51, 256)
    ni50 = idx51[:, :, 1:51].reshape(-1)
    r2_50 = d51[:, :, 1:51].reshape(M1, 50)
    ni10 = idx51[:, :, 1:11].reshape(-1)
    r2_10 = d51[:, :, 1:11].reshape(M1, 10)
    ni4 = idx51[:, :, 1:5].reshape(-1)

    vf4 = jnp.pad(vf, ((0, 0), (0, 1)))
    vfp1 = jnp.pad(vf, ((0, 0), (0, 125)))
    xyzb = _sc_gather(vfp1, ni50)
    xyz50 = xyzb.reshape(M1, 50, 128)[:, :, :4]
    xyz10 = xyz50[:, :10]

    fm0 = _bn(_conv_surface(xyz50, vf4, r2_50, p["conv0_dir"]),
              p["bn0_g"], p["bn0_b"])
    fm0l = _bn(_conv_surface(xyz10, vf4, r2_10, p["conv0l_dir"]),
               p["bn0l_g"], p["bn0l_b"])
    c1, c1l, _, _ = _conv_layer_pair(
        fm0, fm0l, xyz50, xyz10, vf4, ni50, ni10, r2_50, r2_10,
        p["conv1_w"], p["conv1_b"], p["conv1_dir"],
        p["conv1l_w"], p["conv1l_b"], p["conv1l_dir"])
    fm1 = _bn(c1, p["bn1_g"], p["bn1_b"])
    fm1l = _bn(c1l, p["bn1l_g"], p["bn1l_b"])
    fm1t = _mm(jnp.concatenate([fm1, fm1l], axis=1),
               p["down0_w"], p["down0_b"])

    V1 = V // 4
    perm1 = jax.random.permutation(jax.random.key(1), V)[:V1]
    pooled = _comb_pool(_sc_gather(fm1t, ni4).reshape(M1, 4, 128))
    fp1 = pooled.reshape(bs, V, 128)[:, perm1].reshape(bs * V1, 128)
    vp1 = verts[:, perm1]
    v1f = vp1.reshape(bs * V1, 3)
    M2 = bs * V1

    idx51_2, d51_2 = _topk(vp1, 51, V1)
    n2_50 = idx51_2[:, :, 1:51].reshape(-1)
    q2_50 = d51_2[:, :, 1:51].reshape(M2, 50)
    n2_10 = idx51_2[:, :, 1:11].reshape(-1)
    q2_10 = d51_2[:, :, 1:11].reshape(M2, 10)
    n2_4 = idx51_2[:, :, 1:5].reshape(-1)

    v1f4 = jnp.pad(v1f, ((0, 0), (0, 1)))
    vfp2 = jnp.pad(v1f, ((0, 0), (0, 125)))
    c2, c2l, xyz2_50, xyz2_10 = _conv_layer_pair(
        fp1, fp1, None, None, v1f4, n2_50, n2_10, q2_50, q2_10,
        p["conv2_w"], p["conv2_b"], p["conv2_dir"],
        p["conv2l_w"], p["conv2l_b"], p["conv2l_dir"], vfp=vfp2)
    fm2 = _bn(c2, p["bn2_g"], p["bn2_b"])
    fm2l = _bn(c2l, p["bn2l_g"], p["bn2l_b"])
    c3, c3l, _, _ = _conv_layer_pair(
        fm2, fm2l, xyz2_50, xyz2_10, v1f4, n2_50, n2_10, q2_50, q2_10,
        p["conv3_w"], p["conv3_b"], p["conv3_dir"],
        p["conv3l_w"], p["conv3l_b"], p["conv3l_dir"])
    fm3 = _bn(c3, p["bn3_g"], p["bn3_b"])
    fm3l = _bn(c3l, p["bn3l_g"], p["bn3l_b"])
    fm3t = _mm(jnp.concatenate([fm3, fm3l], axis=1),
               p["down1_w"], p["down1_b"])

    V2 = V1 // 4
    perm2 = jax.random.permutation(jax.random.key(2), V1)[:V2]
    pooled2 = _comb_pool(_sc_gather(fm3t, n2_4).reshape(M2, 4, 256))
    fp2 = pooled2.reshape(bs, V1, 256)[:, perm2].reshape(bs * V2, 256)
    vp2 = vp1[:, perm2]
    v2f = vp2.reshape(bs * V2, 3)
    M3 = bs * V2

    idx51_3, d51_3 = _topk(vp2, 51, V2)
    n3_50 = idx51_3[:, :, 1:51].reshape(-1)
    q3_50 = d51_3[:, :, 1:51].reshape(M3, 50)
    n3_10 = idx51_3[:, :, 1:11].reshape(-1)
    q3_10 = d51_3[:, :, 1:11].reshape(M3, 10)

    v2f4 = jnp.pad(v2f, ((0, 0), (0, 1)))
    vfp3 = jnp.pad(v2f, ((0, 0), (0, 125)))
    fm4, fm4l, _, _ = _conv_layer_pair(
        fp2, fp2, None, None, v2f4, n3_50, n3_10, q3_50, q3_10,
        p["conv4_w"], p["conv4_b"], p["conv4_dir"],
        p["conv4l_w"], p["conv4l_b"], p["conv4l_dir"], vfp=vfp3)
    fm4t = _mm(jnp.concatenate([fm4, fm4l], axis=1),
               p["down2_w"], p["down2_b"])

    fglob = _rowmax(fm4t.reshape(bs, V2, 512))

    np1 = _nearest(vp1, verts).reshape(-1)
    np2 = _nearest(vp2, verts).reshape(-1)
    tabn = jnp.concatenate([jnp.pad(fm3t, ((0, 0), (0, 256))), fm4t], axis=0)
    gn = _sc_gather(tabn, jnp.concatenate([np1, np2 + M2]))
    fm3f = gn[:M1, :256]
    fm4f = gn[M1:]
    fg = jnp.broadcast_to(fglob[:, None, :], (bs, V, 512)).reshape(M1, 512)
    oh = jnp.broadcast_to(onehot[:, None, :],
                          (bs, V, onehot.shape[1])).reshape(M1, -1)
    fuse = jnp.concatenate([fm1t, fm3f, fm4f, fg, oh], axis=1)

    x = _mm(fuse, p["h1_w"].T, p["h1_b"], act="relu")
    x = _mm(x, p["h2_w"].T, p["h2_b"], act="relu")
    x = _mm(x, p["h3_w"].T, p["h3_b"])
    return _lsm(x).reshape(bs, V, 50)
